# Initial kernel scaffold; baseline (speedup 1.0000x reference)
import jax
import jax.numpy as jnp
from jax.experimental import pallas as pl


def kernel(x, edge_index, edge_attr, batch, lin1_W, lin1_b, gate_lin1_W, gate_lin2_W, gate_att_l, gate_att_r, gate_bias, gru0_Wih, gru0_Whh, gru0_bih, gru0_bhh, gat_W, gat_att_src, gat_att_dst, gat_bias, gru1_Wih, gru1_Whh, gru1_bih, gru1_bhh, mol_W, mol_att_src, mol_att_dst, mol_bias, mgru_Wih, mgru_Whh, mgru_bih, mgru_bhh, head_W1, head_b1, head_W2, head_b2):
    raise NotImplementedError("write your pallas kernel here")


# speedup vs baseline: 5.3373x; 5.3373x over previous
import functools

import jax
import jax.numpy as jnp
from jax import lax
from jax.experimental import pallas as pl
from jax.experimental.pallas import tpu as pltpu
from jax.experimental.pallas import tpu_sc as plsc

N0 = 10000
NP = 10240
E = 320000
FIN = 128
H = 200
HP = 256
HH = 128
H3P = 768
ED = 16
G = 512
NT = 12
NBLK = NP // 256
EBLK = E // 512

NC = 2
NS = 16
C = 80
EPT = E // NS
EPW = E // (NC * NS)
NPT = NP // NS

@functools.cache
def _mesh():
    return plsc.VectorSubcoreMesh(core_axis_name="c", subcore_axis_name="s",
                                  num_cores=NC, num_subcores=NS)


def _leaky(v):
    return jnp.where(v > 0, v, 0.01 * v)


def _elu(v):
    return jnp.where(v > 0, v, jnp.exp(v) - 1.0)


@functools.cache
def _sc_gather_rows_k():
    @functools.partial(
        pl.kernel,
        out_type=jax.ShapeDtypeStruct((2 * E, HH), jnp.float32),
        mesh=_mesh(),
        compiler_params=pltpu.CompilerParams(needs_layout_passes=False, use_tc_tiling_on_sc=False),
        scratch_types=[
            pltpu.VMEM((C,), jnp.int32),
            pltpu.VMEM((C,), jnp.int32),
            pltpu.VMEM((C, HH), jnp.float32),
        ],
    )
    def k(tab_hbm, idx_hbm, out_hbm, iraw, iadj, rbuf):
        c = lax.axis_index("c")
        s = lax.axis_index("s")
        t0 = s * EPT
        off = c * NP

        def body(g, carry):
            e0 = t0 + g * C
            pltpu.sync_copy(idx_hbm.at[pl.ds(e0, C)], iraw)
            for j in range(C // 16):
                iadj[pl.ds(j * 16, 16)] = iraw[pl.ds(j * 16, 16)] + off
            pltpu.sync_copy(tab_hbm.at[iadj], rbuf)
            pltpu.sync_copy(rbuf, out_hbm.at[pl.ds(c * E + e0, C)])
            return carry

        lax.fori_loop(0, EPT // C, body, 0)

    return k


def _sc_gather_rows(tabf, idx):
    return _sc_gather_rows_k()(tabf, idx)


@functools.cache
def _sc_scatter_rows_k():
    @functools.partial(
        pl.kernel,
        out_type=jax.ShapeDtypeStruct((2 * NP, HH), jnp.float32),
        mesh=_mesh(),
        compiler_params=pltpu.CompilerParams(needs_layout_passes=False, use_tc_tiling_on_sc=False),
        scratch_types=[
            pltpu.VMEM((C,), jnp.int32),
            pltpu.VMEM((C, HH), jnp.float32),
            pltpu.VMEM_SHARED((NP, HH), jnp.float32),
        ],
    )
    def k(rows_hbm, idx_hbm, zero_hbm, out_hbm, ibuf, rbuf, acc):
        c = lax.axis_index("c")
        s = lax.axis_index("s")
        pltpu.sync_copy(zero_hbm, acc.at[pl.ds(s * NPT, NPT)])
        plsc.subcore_barrier()

        t0 = s * EPT

        def body(g, carry):
            e0 = t0 + g * C
            pltpu.sync_copy(idx_hbm.at[pl.ds(e0, C)], ibuf)
            pltpu.sync_copy(rows_hbm.at[pl.ds(c * E + e0, C)], rbuf)
            pltpu.sync_copy(rbuf, acc.at[ibuf], add=True)
            return carry

        lax.fori_loop(0, EPT // C, body, 0)
        plsc.subcore_barrier()
        pltpu.sync_copy(acc.at[pl.ds(s * NPT, NPT)],
                        out_hbm.at[pl.ds(c * NP + s * NPT, NPT)])

    return k


def _sc_scatter_rows(rowsf, idx, zero_tile):
    return _sc_scatter_rows_k()(rowsf, idx, zero_tile)


@functools.cache
def _sc_gather_scal_k():
    @functools.partial(
        pl.kernel,
        out_type=jax.ShapeDtypeStruct((E,), jnp.float32),
        mesh=_mesh(),
        compiler_params=pltpu.CompilerParams(needs_layout_passes=False, use_tc_tiling_on_sc=False),
        scratch_types=[
            pltpu.VMEM((NP // 16, 16), jnp.float32),
            pltpu.VMEM((C,), jnp.int32),
            pltpu.VMEM((C,), jnp.float32),
        ],
    )
    def k(tab_hbm, idx_hbm, out_hbm, tv, ibuf, obuf):
        c = lax.axis_index("c")
        s = lax.axis_index("s")
        w = s * NC + c
        pltpu.sync_copy(tab_hbm, tv)
        t0 = w * EPW

        def body(g, carry):
            e0 = t0 + g * C
            pltpu.sync_copy(idx_hbm.at[pl.ds(e0, C)], ibuf)
            for j in range(C // 16):
                v = ibuf[pl.ds(j * 16, 16)]
                obuf[pl.ds(j * 16, 16)] = plsc.load_gather(
                    tv, [lax.shift_right_logical(v, 4), v & 15])
            pltpu.sync_copy(obuf, out_hbm.at[pl.ds(e0, C)])
            return carry

        lax.fori_loop(0, EPW // C, body, 0)

    return k


def _sc_gather_scal(tab, idx):
    return _sc_gather_scal_k()(tab.reshape(NP // 16, 16), idx)


@functools.cache
def _sc_edge_w_k():
    @functools.partial(
        pl.kernel,
        out_type=jax.ShapeDtypeStruct((E,), jnp.float32),
        mesh=_mesh(),
        compiler_params=pltpu.CompilerParams(needs_layout_passes=False, use_tc_tiling_on_sc=False),
        scratch_types=[
            pltpu.VMEM((NP // 16, 16), jnp.float32),
            pltpu.VMEM((NP // 16, 16), jnp.float32),
            pltpu.VMEM((C,), jnp.int32),
            pltpu.VMEM((C,), jnp.int32),
            pltpu.VMEM((C,), jnp.float32),
        ],
    )
    def k(sa_hbm, sb_hbm, src_hbm, dst_hbm, out_hbm, av, bv, sbuf, dbuf, obuf):
        c = lax.axis_index("c")
        s = lax.axis_index("s")
        w = s * NC + c
        pltpu.sync_copy(sa_hbm, av)
        pltpu.sync_copy(sb_hbm, bv)
        t0 = w * EPW

        def body(g, carry):
            e0 = t0 + g * C
            pltpu.sync_copy(src_hbm.at[pl.ds(e0, C)], sbuf)
            pltpu.sync_copy(dst_hbm.at[pl.ds(e0, C)], dbuf)
            for j in range(C // 16):
                vs = sbuf[pl.ds(j * 16, 16)]
                vd = dbuf[pl.ds(j * 16, 16)]
                va = plsc.load_gather(
                    av, [lax.shift_right_logical(vs, 4), vs & 15])
                vb = plsc.load_gather(
                    bv, [lax.shift_right_logical(vd, 4), vd & 15])
                x = va + vb
                l = jnp.where(x > 0, x, 0.01 * x)
                obuf[pl.ds(j * 16, 16)] = jnp.exp(l)
            pltpu.sync_copy(obuf, out_hbm.at[pl.ds(e0, C)])
            return carry

        lax.fori_loop(0, EPW // C, body, 0)

    return k


def _sc_edge_w(sa, sb, src, dst):
    return _sc_edge_w_k()(sa.reshape(NP // 16, 16), sb.reshape(NP // 16, 16),
                          src, dst)


def _dot(a, b):
    return jnp.dot(a, b, preferred_element_type=jnp.float32)


def _gru(x, h, WihT, WhhT, bih, bhh):
    gi = _dot(x, WihT) + bih
    gh = _dot(h, WhhT) + bhh
    i_r, i_z, i_n = gi[:, :HP], gi[:, HP:2 * HP], gi[:, 2 * HP:]
    h_r, h_z, h_n = gh[:, :HP], gh[:, HP:2 * HP], gh[:, 2 * HP:]
    r = jax.nn.sigmoid(i_r + h_r)
    z = jax.nn.sigmoid(i_z + h_z)
    n = jnp.tanh(i_n + r * h_n)
    return (1.0 - z) * n + z * h


def _tc_node_pre_body(x_ref, W1T_ref, b1_ref, WaT_ref, attr_ref,
                      h0_ref, A2_ref, d_ref):
    x = x_ref[...]
    h0 = _leaky(_dot(x, W1T_ref[...]) + b1_ref[...])
    h0_ref[...] = h0
    A = _dot(h0, WaT_ref[...])
    A2_ref[0] = A[:, :HH]
    A2_ref[1] = A[:, HH:]
    d_ref[0, 0, :] = jnp.sum(h0 * attr_ref[...], axis=1)


def _tc_node_pre(x, W1T, b1, WaT, attr):
    return pl.pallas_call(
        _tc_node_pre_body,
        grid=(NBLK,),
        in_specs=[
            pl.BlockSpec((256, FIN), lambda i: (i, 0)),
            pl.BlockSpec((FIN, HP), lambda i: (0, 0)),
            pl.BlockSpec((1, HP), lambda i: (0, 0)),
            pl.BlockSpec((HP, HP), lambda i: (0, 0)),
            pl.BlockSpec((1, HP), lambda i: (0, 0)),
        ],
        out_specs=[
            pl.BlockSpec((256, HP), lambda i: (i, 0)),
            pl.BlockSpec((2, 256, HH), lambda i: (0, i, 0)),
            pl.BlockSpec((1, 1, 256), lambda i: (i, 0, 0)),
        ],
        out_shape=[
            jax.ShapeDtypeStruct((NP, HP), jnp.float32),
            jax.ShapeDtypeStruct((2, NP, HH), jnp.float32),
            jax.ShapeDtypeStruct((NBLK, 1, 256), jnp.float32),
        ],
    )(x, W1T, b1, WaT, attr)


def _tc_edge1_body(Ag_ref, ea_ref, dg_ref, WbT_ref, attl_ref, P_ref):
    Eb = _dot(ea_ref[...], WbT_ref[...])
    Ag = jnp.concatenate([Ag_ref[0], Ag_ref[1]], axis=1)
    m = _leaky(Ag + Eb)
    l = _leaky(jnp.sum(m * attl_ref[...], axis=1) + dg_ref[0, 0, :])
    w = jnp.exp(l)
    P = m * w[:, None]
    col = lax.broadcasted_iota(jnp.int32, (512, HP), 1)
    P = jnp.where(col == H, w[:, None], P)
    P_ref[0] = P[:, :HH]
    P_ref[1] = P[:, HH:]


def _tc_edge1(Ag2, ea, dg, WbT, attl):
    return pl.pallas_call(
        _tc_edge1_body,
        grid=(EBLK,),
        in_specs=[
            pl.BlockSpec((2, 512, HH), lambda i: (0, i, 0)),
            pl.BlockSpec((512, ED), lambda i: (i, 0)),
            pl.BlockSpec((1, 1, 512), lambda i: (i, 0, 0)),
            pl.BlockSpec((ED, HP), lambda i: (0, 0)),
            pl.BlockSpec((1, HP), lambda i: (0, 0)),
        ],
        out_specs=pl.BlockSpec((2, 512, HH), lambda i: (0, i, 0)),
        out_shape=jax.ShapeDtypeStruct((2, E, HH), jnp.float32),
    )(Ag2, ea, dg, WbT, attl)


def _tc_node_mid_body(S2_ref, h0_ref, W2T_ref, gb_ref, WihT_ref, WhhT_ref,
                      bih_ref, bhh_ref, WgT_ref, asrc_ref, adst_ref,
                      xc_ref, xt2_ref, sa_ref, sb_ref):
    M = jnp.concatenate([S2_ref[0], S2_ref[1]], axis=1)
    s = M[:, H]
    h = _elu(_dot(M, W2T_ref[...]) / (s[:, None] + 1e-16) + gb_ref[...])
    h0 = h0_ref[...]
    xc = jnp.maximum(_gru(h, h0, WihT_ref[...], WhhT_ref[...],
                          bih_ref[...], bhh_ref[...]), 0.0)
    xc_ref[...] = xc
    xt = _dot(xc, WgT_ref[...])
    xt2_ref[0] = xt[:, :HH]
    xt2_ref[1] = xt[:, HH:]
    sa_ref[0, 0, :] = jnp.sum(xt * asrc_ref[...], axis=1)
    sb_ref[0, 0, :] = jnp.sum(xt * adst_ref[...], axis=1)


def _tc_node_mid(S2, h0, W2T, gb, WihT, WhhT, bih, bhh, WgT, asrc, adst):
    return pl.pallas_call(
        _tc_node_mid_body,
        grid=(NBLK,),
        in_specs=[
            pl.BlockSpec((2, 256, HH), lambda i: (0, i, 0)),
            pl.BlockSpec((256, HP), lambda i: (i, 0)),
            pl.BlockSpec((HP, HP), lambda i: (0, 0)),
            pl.BlockSpec((1, HP), lambda i: (0, 0)),
            pl.BlockSpec((HP, H3P), lambda i: (0, 0)),
            pl.BlockSpec((HP, H3P), lambda i: (0, 0)),
            pl.BlockSpec((1, H3P), lambda i: (0, 0)),
            pl.BlockSpec((1, H3P), lambda i: (0, 0)),
            pl.BlockSpec((HP, HP), lambda i: (0, 0)),
            pl.BlockSpec((1, HP), lambda i: (0, 0)),
            pl.BlockSpec((1, HP), lambda i: (0, 0)),
        ],
        out_specs=[
            pl.BlockSpec((256, HP), lambda i: (i, 0)),
            pl.BlockSpec((2, 256, HH), lambda i: (0, i, 0)),
            pl.BlockSpec((1, 1, 256), lambda i: (i, 0, 0)),
            pl.BlockSpec((1, 1, 256), lambda i: (i, 0, 0)),
        ],
        out_shape=[
            jax.ShapeDtypeStruct((NP, HP), jnp.float32),
            jax.ShapeDtypeStruct((2, NP, HH), jnp.float32),
            jax.ShapeDtypeStruct((NBLK, 1, 256), jnp.float32),
            jax.ShapeDtypeStruct((NBLK, 1, 256), jnp.float32),
        ],
    )(S2, h0, W2T, gb, WihT, WhhT, bih, bhh, WgT, asrc, adst)


def _tc_edge2_body(Xg_ref, w_ref, P_ref):
    Xg = jnp.concatenate([Xg_ref[0], Xg_ref[1]], axis=1)
    w = w_ref[0, 0, :]
    P = Xg * w[:, None]
    col = lax.broadcasted_iota(jnp.int32, (512, HP), 1)
    P = jnp.where(col == H, w[:, None], P)
    P_ref[0] = P[:, :HH]
    P_ref[1] = P[:, HH:]


def _tc_edge2(Xg2, w):
    return pl.pallas_call(
        _tc_edge2_body,
        grid=(EBLK,),
        in_specs=[
            pl.BlockSpec((2, 512, HH), lambda i: (0, i, 0)),
            pl.BlockSpec((1, 1, 512), lambda i: (i, 0, 0)),
        ],
        out_specs=pl.BlockSpec((2, 512, HH), lambda i: (0, i, 0)),
        out_shape=jax.ShapeDtypeStruct((2, E, HH), jnp.float32),
    )(Xg2, w)


def _tc_node_post_body(X2_ref, xc_ref, gb_ref, WihT_ref, WhhT_ref, bih_ref,
                       bhh_ref, WmT_ref, asrc_ref, xc2_ref, xs_ref, an_ref):
    X = jnp.concatenate([X2_ref[0], X2_ref[1]], axis=1)
    s = X[:, H]
    h = _elu(X / (s[:, None] + 1e-16) + gb_ref[...])
    xc = xc_ref[...]
    xc2 = jnp.maximum(_gru(h, xc, WihT_ref[...], WhhT_ref[...],
                           bih_ref[...], bhh_ref[...]), 0.0)
    xc2_ref[...] = xc2
    xs = _dot(xc2, WmT_ref[...])
    xs_ref[...] = xs
    an_ref[0, 0, :] = jnp.sum(xs * asrc_ref[...], axis=1)


def _tc_node_post(X2, xc, gb, WihT, WhhT, bih, bhh, WmT, asrc):
    return pl.pallas_call(
        _tc_node_post_body,
        grid=(NBLK,),
        in_specs=[
            pl.BlockSpec((2, 256, HH), lambda i: (0, i, 0)),
            pl.BlockSpec((256, HP), lambda i: (i, 0)),
            pl.BlockSpec((1, HP), lambda i: (0, 0)),
            pl.BlockSpec((HP, H3P), lambda i: (0, 0)),
            pl.BlockSpec((HP, H3P), lambda i: (0, 0)),
            pl.BlockSpec((1, H3P), lambda i: (0, 0)),
            pl.BlockSpec((1, H3P), lambda i: (0, 0)),
            pl.BlockSpec((HP, HP), lambda i: (0, 0)),
            pl.BlockSpec((1, HP), lambda i: (0, 0)),
        ],
        out_specs=[
            pl.BlockSpec((256, HP), lambda i: (i, 0)),
            pl.BlockSpec((256, HP), lambda i: (i, 0)),
            pl.BlockSpec((1, 1, 256), lambda i: (i, 0, 0)),
        ],
        out_shape=[
            jax.ShapeDtypeStruct((NP, HP), jnp.float32),
            jax.ShapeDtypeStruct((NP, HP), jnp.float32),
            jax.ShapeDtypeStruct((NBLK, 1, 256), jnp.float32),
        ],
    )(X2, xc, gb, WihT, WhhT, bih, bhh, WmT, asrc)


def _tc_seg_sum_body(xc2_ref, b_ref, o_ref):
    i = pl.program_id(0)
    b = b_ref[0, 0, :]
    oh = (b[:, None] == lax.broadcasted_iota(jnp.int32, (256, G), 1)
          ).astype(jnp.float32)
    contrib = lax.dot_general(oh, xc2_ref[...], (((0,), (0,)), ((), ())),
                              preferred_element_type=jnp.float32)
    prev = jnp.where(i == 0, jnp.zeros_like(contrib), o_ref[...])
    acc = prev + contrib
    o_ref[...] = jnp.where(i == NBLK - 1, jnp.maximum(acc, 0.0), acc)


def _tc_seg_sum(xc2, batch3):
    return pl.pallas_call(
        _tc_seg_sum_body,
        grid=(NBLK,),
        in_specs=[
            pl.BlockSpec((256, HP), lambda i: (i, 0)),
            pl.BlockSpec((1, 1, 256), lambda i: (i, 0, 0)),
        ],
        out_specs=pl.BlockSpec((G, HP), lambda i: (0, 0)),
        out_shape=jax.ShapeDtypeStruct((G, HP), jnp.float32),
    )(xc2, batch3)


def _tc_mol_iter_body(out_ref, xs_ref, an_ref, b_ref, WmT_ref, adst_ref,
                      mb_ref, WihT_ref, WhhT_ref, bih_ref, bhh_ref,
                      onew_ref, dd_scr, s3_scr, Hm_scr):
    i = pl.program_id(0)

    @pl.when(i == 0)
    def _():
        od = _dot(out_ref[...], WmT_ref[...])
        dd_scr[0, :] = jnp.sum(od * adst_ref[...], axis=1)
        s3_scr[...] = jnp.zeros_like(s3_scr)
        Hm_scr[...] = jnp.zeros_like(Hm_scr)

    b = b_ref[0, 0, :]
    oh = (b[:, None] == lax.broadcasted_iota(jnp.int32, (256, G), 1)
          ).astype(jnp.float32)
    ddb = jnp.sum(oh * dd_scr[0, :][None, :], axis=1)
    l3 = _leaky(an_ref[0, 0, :] + ddb)
    e3 = jnp.exp(l3)
    s3_scr[0, :] += jnp.sum(oh * e3[:, None], axis=0)
    Hm_scr[...] += lax.dot_general(oh, xs_ref[...] * e3[:, None],
                                   (((0,), (0,)), ((), ())),
                                   preferred_element_type=jnp.float32)

    @pl.when(i == NBLK - 1)
    def _():
        s3 = s3_scr[0, :]
        h = _elu(Hm_scr[...] / (s3[:, None] + 1e-16) + mb_ref[...])
        o = out_ref[...]
        onew = _gru(h, o, WihT_ref[...], WhhT_ref[...],
                    bih_ref[...], bhh_ref[...])
        onew_ref[...] = jnp.maximum(onew, 0.0)


def _tc_mol_iter(out, xs, an, batch3, WmT, adst, mb, WihT, WhhT, bih, bhh):
    return pl.pallas_call(
        _tc_mol_iter_body,
        grid=(NBLK,),
        in_specs=[
            pl.BlockSpec((G, HP), lambda i: (0, 0)),
            pl.BlockSpec((256, HP), lambda i: (i, 0)),
            pl.BlockSpec((1, 1, 256), lambda i: (i, 0, 0)),
            pl.BlockSpec((1, 1, 256), lambda i: (i, 0, 0)),
            pl.BlockSpec((HP, HP), lambda i: (0, 0)),
            pl.BlockSpec((1, HP), lambda i: (0, 0)),
            pl.BlockSpec((1, HP), lambda i: (0, 0)),
            pl.BlockSpec((HP, H3P), lambda i: (0, 0)),
            pl.BlockSpec((HP, H3P), lambda i: (0, 0)),
            pl.BlockSpec((1, H3P), lambda i: (0, 0)),
            pl.BlockSpec((1, H3P), lambda i: (0, 0)),
        ],
        out_specs=pl.BlockSpec((G, HP), lambda i: (0, 0)),
        out_shape=jax.ShapeDtypeStruct((G, HP), jnp.float32),
        scratch_shapes=[
            pltpu.VMEM((1, G), jnp.float32),
            pltpu.VMEM((1, G), jnp.float32),
            pltpu.VMEM((G, HP), jnp.float32),
        ],
    )(out, xs, an, batch3, WmT, adst, mb, WihT, WhhT, bih, bhh)


def _tc_head_body(out_ref, W1T_ref, b1_ref, W2_ref, b2_ref, o_ref):
    h1 = jnp.maximum(_dot(out_ref[...], W1T_ref[...]) + b1_ref[...], 0.0)
    o_ref[...] = _dot(h1, W2_ref[...]) + b2_ref[...]


def _tc_head(out, W1T, b1, W2blk, b2):
    return pl.pallas_call(
        _tc_head_body,
        grid=(1,),
        in_specs=[
            pl.BlockSpec((G, HP), lambda i: (0, 0)),
            pl.BlockSpec((HP, 1280), lambda i: (0, 0)),
            pl.BlockSpec((1, 1280), lambda i: (0, 0)),
            pl.BlockSpec((1280, 128), lambda i: (0, 0)),
            pl.BlockSpec((1, 128), lambda i: (0, 0)),
        ],
        out_specs=pl.BlockSpec((G, 128), lambda i: (0, 0)),
        out_shape=jax.ShapeDtypeStruct((G, 128), jnp.float32),
    )(out, W1T, b1, W2blk, b2)


def _padT(W, rows, cols):
    r0, c0 = W.shape
    Wp = jnp.zeros((rows, cols), W.dtype).at[:r0, :c0].set(W)
    return Wp.T


def _padv(v, n):
    return jnp.zeros((1, n), v.dtype).at[0, :v.shape[0]].set(v)


def _pad_gru(Wih, Whh, bih, bhh):
    WihT = jnp.zeros((H3P, HP), Wih.dtype)
    WhhT = jnp.zeros((H3P, HP), Whh.dtype)
    bihp = jnp.zeros((1, H3P), bih.dtype)
    bhhp = jnp.zeros((1, H3P), bhh.dtype)
    for k in range(3):
        WihT = WihT.at[k * HP:k * HP + H, :H].set(Wih[k * H:(k + 1) * H])
        WhhT = WhhT.at[k * HP:k * HP + H, :H].set(Whh[k * H:(k + 1) * H])
        bihp = bihp.at[0, k * HP:k * HP + H].set(bih[k * H:(k + 1) * H])
        bhhp = bhhp.at[0, k * HP:k * HP + H].set(bhh[k * H:(k + 1) * H])
    return WihT.T, WhhT.T, bihp, bhhp


def kernel(x, edge_index, edge_attr, batch, lin1_W, lin1_b, gate_lin1_W,
           gate_lin2_W, gate_att_l, gate_att_r, gate_bias, gru0_Wih, gru0_Whh,
           gru0_bih, gru0_bhh, gat_W, gat_att_src, gat_att_dst, gat_bias,
           gru1_Wih, gru1_Whh, gru1_bih, gru1_bhh, mol_W, mol_att_src,
           mol_att_dst, mol_bias, mgru_Wih, mgru_Whh, mgru_bih, mgru_bhh,
           head_W1, head_b1, head_W2, head_b2):
    src = edge_index[0]
    dst = edge_index[1]
    xp = jnp.zeros((NP, FIN), jnp.float32).at[:N0].set(x)
    batchp = jnp.full((NP,), G, jnp.int32).at[:N0].set(batch)
    batch3 = batchp.reshape(NBLK, 1, 256)

    W1T = _padT(lin1_W, HP, FIN)
    b1 = _padv(lin1_b, HP)
    WaT = _padT(gate_lin1_W[:, :H], HP, HP)
    WbT = _padT(gate_lin1_W[:, H:], HP, ED)
    attl = _padv(gate_att_l, HP)
    attr_ = _padv(gate_att_r, HP)
    W2T = _padT(gate_lin2_W, HP, HP)
    gb = _padv(gate_bias, HP)
    g0 = _pad_gru(gru0_Wih, gru0_Whh, gru0_bih, gru0_bhh)
    g1 = _pad_gru(gru1_Wih, gru1_Whh, gru1_bih, gru1_bhh)
    gm = _pad_gru(mgru_Wih, mgru_Whh, mgru_bih, mgru_bhh)
    WgT = _padT(gat_W, HP, HP)
    gasrc = _padv(gat_att_src, HP)
    gadst = _padv(gat_att_dst, HP)
    gatb = _padv(gat_bias, HP)
    WmT = _padT(mol_W, HP, HP)
    masrc = _padv(mol_att_src, HP)
    madst = _padv(mol_att_dst, HP)
    mb = _padv(mol_bias, HP)
    W1r = head_W1.reshape(NT * (H // 2), H)
    hW1T = _padT(W1r, 1280, HP)
    hb1 = _padv(head_b1.reshape(-1), 1280)
    W2blk = jnp.zeros((1280, 128), jnp.float32)
    for k in range(NT):
        W2blk = W2blk.at[k * (H // 2):(k + 1) * (H // 2), k].set(head_W2[k, 0])
    hb2 = _padv(head_b2[:, 0], 128)

    zero_tile = jnp.zeros((NPT, HH), jnp.float32)

    h0, A2, d3 = _tc_node_pre(xp, W1T, b1, WaT, attr_)

    Ag2 = _sc_gather_rows(A2.reshape(2 * NP, HH), src)
    dg = _sc_gather_scal(d3.reshape(NP), dst)
    P2 = _tc_edge1(Ag2.reshape(2, E, HH), edge_attr,
                   dg.reshape(EBLK, 1, 512), WbT, attl)
    S2 = _sc_scatter_rows(P2.reshape(2 * E, HH), dst, zero_tile)

    xc, xt2, sa3, sb3 = _tc_node_mid(S2.reshape(2, NP, HH), h0, W2T, gb,
                                     *g0, WgT, gasrc, gadst)

    w2 = _sc_edge_w(sa3.reshape(NP), sb3.reshape(NP), src, dst)
    Xg2 = _sc_gather_rows(xt2.reshape(2 * NP, HH), src)
    Q2 = _tc_edge2(Xg2.reshape(2, E, HH), w2.reshape(EBLK, 1, 512))
    X2 = _sc_scatter_rows(Q2.reshape(2 * E, HH), dst, zero_tile)

    xc2, xs, an3 = _tc_node_post(X2.reshape(2, NP, HH), xc, gatb,
                                 *g1, WmT, masrc)

    out = _tc_seg_sum(xc2, batch3)
    for _ in range(2):
        out = _tc_mol_iter(out, xs, an3, batch3, WmT, madst, mb, *gm)

    logits = _tc_head(out, hW1T, hb1, W2blk, hb2)
    return logits[:, :NT]


# speedup vs baseline: 8.2626x; 1.5481x over previous
import functools

import jax
import jax.numpy as jnp
from jax import lax
from jax.experimental import pallas as pl
from jax.experimental.pallas import tpu as pltpu
from jax.experimental.pallas import tpu_sc as plsc

N0 = 10000
NP = 10240
E = 320000
FIN = 128
H = 200
HP = 256
HH = 128
H3P = 768
ED = 16
G = 512
NT = 12
NBLK = NP // 256
EBLK = E // 512

NC = 2
NS = 16
C = 80
EPT = E // NS
EPW = E // (NC * NS)
NPT = NP // NS

@functools.cache
def _mesh():
    return plsc.VectorSubcoreMesh(core_axis_name="c", subcore_axis_name="s",
                                  num_cores=NC, num_subcores=NS)


def _leaky(v):
    return jnp.where(v > 0, v, 0.01 * v)


def _elu(v):
    return jnp.where(v > 0, v, jnp.exp(v) - 1.0)


@functools.cache
def _sc_gather_rows_k():
    @functools.partial(
        pl.kernel,
        out_type=jax.ShapeDtypeStruct((2 * E, HH), jnp.float32),
        mesh=_mesh(),
        compiler_params=pltpu.CompilerParams(needs_layout_passes=False, use_tc_tiling_on_sc=False),
        scratch_types=[
            pltpu.VMEM((C,), jnp.int32),
            pltpu.VMEM((C,), jnp.int32),
            pltpu.VMEM((C, HH), jnp.float32),
        ],
    )
    def k(tab_hbm, idx_hbm, out_hbm, iraw, iadj, rbuf):
        c = lax.axis_index("c")
        s = lax.axis_index("s")
        t0 = s * EPT
        off = c * NP

        def body(g, carry):
            e0 = t0 + g * C
            pltpu.sync_copy(idx_hbm.at[pl.ds(e0, C)], iraw)
            for j in range(C // 16):
                iadj[pl.ds(j * 16, 16)] = iraw[pl.ds(j * 16, 16)] + off
            pltpu.sync_copy(tab_hbm.at[iadj], rbuf)
            pltpu.sync_copy(rbuf, out_hbm.at[pl.ds(c * E + e0, C)])
            return carry

        lax.fori_loop(0, EPT // C, body, 0)

    return k


def _sc_gather_rows(tabf, idx):
    return _sc_gather_rows_k()(tabf, idx)


@functools.cache
def _sc_scatter_rows_k():
    @functools.partial(
        pl.kernel,
        out_type=jax.ShapeDtypeStruct((2 * NP, HH), jnp.float32),
        mesh=_mesh(),
        compiler_params=pltpu.CompilerParams(needs_layout_passes=False, use_tc_tiling_on_sc=False),
        scratch_types=[
            pltpu.VMEM((C,), jnp.int32),
            pltpu.VMEM((C, HH), jnp.float32),
            pltpu.VMEM_SHARED((NP, HH), jnp.float32),
        ],
    )
    def k(rows_hbm, idx_hbm, zero_hbm, out_hbm, ibuf, rbuf, acc):
        c = lax.axis_index("c")
        s = lax.axis_index("s")
        pltpu.sync_copy(zero_hbm, acc.at[pl.ds(s * NPT, NPT)])
        plsc.subcore_barrier()

        t0 = s * EPT

        def body(g, carry):
            e0 = t0 + g * C
            pltpu.sync_copy(idx_hbm.at[pl.ds(e0, C)], ibuf)
            pltpu.sync_copy(rows_hbm.at[pl.ds(c * E + e0, C)], rbuf)
            pltpu.sync_copy(rbuf, acc.at[ibuf], add=True)
            return carry

        lax.fori_loop(0, EPT // C, body, 0)
        plsc.subcore_barrier()
        pltpu.sync_copy(acc.at[pl.ds(s * NPT, NPT)],
                        out_hbm.at[pl.ds(c * NP + s * NPT, NPT)])

    return k


def _sc_scatter_rows(rowsf, idx, zero_tile):
    return _sc_scatter_rows_k()(rowsf, idx, zero_tile)


@functools.cache
def _sc_gather_scal_k():
    @functools.partial(
        pl.kernel,
        out_type=jax.ShapeDtypeStruct((E,), jnp.float32),
        mesh=_mesh(),
        compiler_params=pltpu.CompilerParams(needs_layout_passes=False, use_tc_tiling_on_sc=False),
        scratch_types=[
            pltpu.VMEM((NP // 16, 16), jnp.float32),
            pltpu.VMEM((C,), jnp.int32),
            pltpu.VMEM((C,), jnp.float32),
        ],
    )
    def k(tab_hbm, idx_hbm, out_hbm, tv, ibuf, obuf):
        c = lax.axis_index("c")
        s = lax.axis_index("s")
        w = s * NC + c
        pltpu.sync_copy(tab_hbm, tv)
        t0 = w * EPW

        def body(g, carry):
            e0 = t0 + g * C
            pltpu.sync_copy(idx_hbm.at[pl.ds(e0, C)], ibuf)
            for j in range(C // 16):
                v = ibuf[pl.ds(j * 16, 16)]
                obuf[pl.ds(j * 16, 16)] = plsc.load_gather(
                    tv, [lax.shift_right_logical(v, 4), v & 15])
            pltpu.sync_copy(obuf, out_hbm.at[pl.ds(e0, C)])
            return carry

        lax.fori_loop(0, EPW // C, body, 0)

    return k


def _sc_gather_scal(tab, idx):
    return _sc_gather_scal_k()(tab.reshape(NP // 16, 16), idx)


@functools.cache
def _sc_edge_w_k():
    @functools.partial(
        pl.kernel,
        out_type=jax.ShapeDtypeStruct((E,), jnp.float32),
        mesh=_mesh(),
        compiler_params=pltpu.CompilerParams(needs_layout_passes=False, use_tc_tiling_on_sc=False),
        scratch_types=[
            pltpu.VMEM((NP // 16, 16), jnp.float32),
            pltpu.VMEM((NP // 16, 16), jnp.float32),
            pltpu.VMEM((C,), jnp.int32),
            pltpu.VMEM((C,), jnp.int32),
            pltpu.VMEM((C,), jnp.float32),
        ],
    )
    def k(sa_hbm, sb_hbm, src_hbm, dst_hbm, out_hbm, av, bv, sbuf, dbuf, obuf):
        c = lax.axis_index("c")
        s = lax.axis_index("s")
        w = s * NC + c
        pltpu.sync_copy(sa_hbm, av)
        pltpu.sync_copy(sb_hbm, bv)
        t0 = w * EPW

        def body(g, carry):
            e0 = t0 + g * C
            pltpu.sync_copy(src_hbm.at[pl.ds(e0, C)], sbuf)
            pltpu.sync_copy(dst_hbm.at[pl.ds(e0, C)], dbuf)
            for j in range(C // 16):
                vs = sbuf[pl.ds(j * 16, 16)]
                vd = dbuf[pl.ds(j * 16, 16)]
                va = plsc.load_gather(
                    av, [lax.shift_right_logical(vs, 4), vs & 15])
                vb = plsc.load_gather(
                    bv, [lax.shift_right_logical(vd, 4), vd & 15])
                x = va + vb
                l = jnp.where(x > 0, x, 0.01 * x)
                obuf[pl.ds(j * 16, 16)] = jnp.exp(l)
            pltpu.sync_copy(obuf, out_hbm.at[pl.ds(e0, C)])
            return carry

        lax.fori_loop(0, EPW // C, body, 0)

    return k


def _sc_edge_w(sa, sb, src, dst):
    return _sc_edge_w_k()(sa.reshape(NP // 16, 16), sb.reshape(NP // 16, 16),
                          src, dst)


@functools.cache
def _sc_gat_fused_k():
    NCH = EPT // C

    @functools.partial(
        pl.kernel,
        out_type=jax.ShapeDtypeStruct((2 * NP, HH), jnp.float32),
        mesh=_mesh(),
        compiler_params=pltpu.CompilerParams(needs_layout_passes=False,
                                             use_tc_tiling_on_sc=False),
        scratch_types=[
            pltpu.VMEM((NP // 16, 16), jnp.float32),
            pltpu.VMEM((NP // 16, 16), jnp.float32),
            pltpu.VMEM((C,), jnp.int32),
            pltpu.VMEM((C,), jnp.int32),
            pltpu.VMEM((C,), jnp.int32),
            pltpu.VMEM((C,), jnp.int32),
            pltpu.VMEM((C,), jnp.int32),
            pltpu.VMEM((C,), jnp.int32),
            pltpu.VMEM((C,), jnp.int32),
            pltpu.VMEM((C,), jnp.int32),
            pltpu.VMEM((C, HH), jnp.float32),
            pltpu.VMEM((C, HH), jnp.float32),
            pltpu.VMEM((C // 16, 16), jnp.float32),
            pltpu.VMEM_SHARED((NP, HH), jnp.float32),
            pltpu.SemaphoreType.DMA,
            pltpu.SemaphoreType.DMA,
            pltpu.SemaphoreType.DMA,
            pltpu.SemaphoreType.DMA,
            pltpu.SemaphoreType.DMA,
            pltpu.SemaphoreType.DMA,
        ],
    )
    def k(tab_hbm, src_hbm, dst_hbm, sa_hbm, sb_hbm, zero_hbm, out_hbm,
          savm, sbvm, sraw0, sraw1, draw0, draw1, sib0, sib1, dib0, dib1,
          rb0, rb1, obuf, acc, isem0, isem1, gsem0, gsem1, ssem0, ssem1):
        c = lax.axis_index("c")
        s = lax.axis_index("s")
        t0 = s * EPT
        off = c * NP
        lane = lax.iota(jnp.int32, 16)
        is_hi = lax.broadcast(c, (16,)) == 1
        wcol_mask = is_hi & (lane == (H - HH - 64))

        pltpu.sync_copy(sa_hbm, savm)
        pltpu.sync_copy(sb_hbm, sbvm)
        pltpu.sync_copy(zero_hbm, acc.at[pl.ds(s * NPT, NPT)])
        plsc.subcore_barrier()

        def start_idx(g, sraw, draw, isem):
            e0 = t0 + g * C
            pltpu.async_copy(src_hbm.at[pl.ds(e0, C)], sraw, isem)
            pltpu.async_copy(dst_hbm.at[pl.ds(e0, C)], draw, isem)

        def wait_idx(g, sraw, draw, isem):
            e0 = t0 + g * C
            pltpu.make_async_copy(src_hbm.at[pl.ds(e0, C)], sraw, isem).wait()
            pltpu.make_async_copy(dst_hbm.at[pl.ds(e0, C)], draw, isem).wait()

        def fill_sib(sraw, sib):
            for j in range(C // 16):
                sib[pl.ds(j * 16, 16)] = sraw[pl.ds(j * 16, 16)] + off

        def start_gather(sib, rb, gsem):
            pltpu.async_copy(tab_hbm.at[sib], rb, gsem)

        def wait_gather(sib, rb, gsem):
            pltpu.make_async_copy(tab_hbm.at[sib], rb, gsem).wait()

        def start_scatter(rb, dib, ssem):
            pltpu.async_copy(rb, acc.at[dib], ssem, add=True)

        def wait_scatter(rb, dib, ssem):
            pltpu.make_async_copy(rb, acc.at[dib], ssem).wait()

        def process(sraw, draw, dib, rb):
            for j in range(C // 16):
                vs = sraw[pl.ds(j * 16, 16)]
                vd = draw[pl.ds(j * 16, 16)]
                dib[pl.ds(j * 16, 16)] = vd
                va = plsc.load_gather(
                    savm, [lax.shift_right_logical(vs, 4), vs & 15])
                vb = plsc.load_gather(
                    sbvm, [lax.shift_right_logical(vd, 4), vd & 15])
                x = va + vb
                obuf[j] = jnp.exp(jnp.where(x > 0, x, 0.01 * x))

            def row_body(r, carry):
                hi = lax.broadcast(lax.shift_right_logical(r, 4), (16,))
                lo = lax.broadcast(r & 15, (16,))
                wspl = plsc.load_gather(obuf, [hi, lo])
                for kk in range(HH // 16):
                    v = rb[r, pl.ds(kk * 16, 16)] * wspl
                    if kk == 4:
                        v = jnp.where(wcol_mask, wspl, v)
                    rb[r, pl.ds(kk * 16, 16)] = v
                return carry

            lax.fori_loop(0, C, row_body, 0)

        start_idx(0, sraw0, draw0, isem0)
        start_idx(1, sraw1, draw1, isem1)
        wait_idx(0, sraw0, draw0, isem0)
        fill_sib(sraw0, sib0)
        start_gather(sib0, rb0, gsem0)
        wait_idx(1, sraw1, draw1, isem1)
        fill_sib(sraw1, sib1)
        start_gather(sib1, rb1, gsem1)
        wait_gather(sib0, rb0, gsem0)
        process(sraw0, draw0, dib0, rb0)
        start_scatter(rb0, dib0, ssem0)
        start_idx(2, sraw0, draw0, isem0)
        wait_idx(2, sraw0, draw0, isem0)
        fill_sib(sraw0, sib0)
        wait_scatter(rb0, dib0, ssem0)
        start_gather(sib0, rb0, gsem0)
        wait_gather(sib1, rb1, gsem1)
        process(sraw1, draw1, dib1, rb1)
        start_scatter(rb1, dib1, ssem1)
        start_idx(3, sraw1, draw1, isem1)

        def pair(p, carry):
            g = 2 * p
            wait_idx(g + 1, sraw1, draw1, isem1)
            fill_sib(sraw1, sib1)
            wait_scatter(rb1, dib1, ssem1)
            start_gather(sib1, rb1, gsem1)
            wait_gather(sib0, rb0, gsem0)
            process(sraw0, draw0, dib0, rb0)
            start_scatter(rb0, dib0, ssem0)
            start_idx(g + 2, sraw0, draw0, isem0)
            wait_idx(g + 2, sraw0, draw0, isem0)
            fill_sib(sraw0, sib0)
            wait_scatter(rb0, dib0, ssem0)
            start_gather(sib0, rb0, gsem0)
            wait_gather(sib1, rb1, gsem1)
            process(sraw1, draw1, dib1, rb1)
            start_scatter(rb1, dib1, ssem1)
            start_idx(g + 3, sraw1, draw1, isem1)
            return carry

        lax.fori_loop(1, NCH // 2 - 1, pair, 0)

        wait_idx(NCH - 1, sraw1, draw1, isem1)
        fill_sib(sraw1, sib1)
        wait_scatter(rb1, dib1, ssem1)
        start_gather(sib1, rb1, gsem1)
        wait_gather(sib0, rb0, gsem0)
        process(sraw0, draw0, dib0, rb0)
        start_scatter(rb0, dib0, ssem0)
        wait_scatter(rb0, dib0, ssem0)
        wait_gather(sib1, rb1, gsem1)
        process(sraw1, draw1, dib1, rb1)
        start_scatter(rb1, dib1, ssem1)
        wait_scatter(rb1, dib1, ssem1)

        plsc.subcore_barrier()
        pltpu.sync_copy(acc.at[pl.ds(s * NPT, NPT)],
                        out_hbm.at[pl.ds(c * NP + s * NPT, NPT)])

    return k


def _sc_gat_fused(tabf, src, dst, sa, sb, zero_tile):
    return _sc_gat_fused_k()(tabf, src, dst, sa.reshape(NP // 16, 16),
                             sb.reshape(NP // 16, 16), zero_tile)


def _dot(a, b):
    return jnp.dot(a, b, preferred_element_type=jnp.float32)


def _gru(x, h, WihT, WhhT, bih, bhh):
    gi = _dot(x, WihT) + bih
    gh = _dot(h, WhhT) + bhh
    i_r, i_z, i_n = gi[:, :HP], gi[:, HP:2 * HP], gi[:, 2 * HP:]
    h_r, h_z, h_n = gh[:, :HP], gh[:, HP:2 * HP], gh[:, 2 * HP:]
    r = jax.nn.sigmoid(i_r + h_r)
    z = jax.nn.sigmoid(i_z + h_z)
    n = jnp.tanh(i_n + r * h_n)
    return (1.0 - z) * n + z * h


def _tc_node_pre_body(x_ref, W1T_ref, b1_ref, WaT_ref, attr_ref,
                      h0_ref, A2_ref, d_ref):
    x = x_ref[...]
    h0 = _leaky(_dot(x, W1T_ref[...]) + b1_ref[...])
    h0_ref[...] = h0
    A = _dot(h0, WaT_ref[...])
    A2_ref[0] = A[:, :HH]
    A2_ref[1] = A[:, HH:]
    d_ref[0, 0, :] = jnp.sum(h0 * attr_ref[...], axis=1)


def _tc_node_pre(x, W1T, b1, WaT, attr):
    return pl.pallas_call(
        _tc_node_pre_body,
        grid=(NBLK,),
        in_specs=[
            pl.BlockSpec((256, FIN), lambda i: (i, 0)),
            pl.BlockSpec((FIN, HP), lambda i: (0, 0)),
            pl.BlockSpec((1, HP), lambda i: (0, 0)),
            pl.BlockSpec((HP, HP), lambda i: (0, 0)),
            pl.BlockSpec((1, HP), lambda i: (0, 0)),
        ],
        out_specs=[
            pl.BlockSpec((256, HP), lambda i: (i, 0)),
            pl.BlockSpec((2, 256, HH), lambda i: (0, i, 0)),
            pl.BlockSpec((1, 1, 256), lambda i: (i, 0, 0)),
        ],
        out_shape=[
            jax.ShapeDtypeStruct((NP, HP), jnp.float32),
            jax.ShapeDtypeStruct((2, NP, HH), jnp.float32),
            jax.ShapeDtypeStruct((NBLK, 1, 256), jnp.float32),
        ],
    )(x, W1T, b1, WaT, attr)


def _tc_edge1_body(Ag_ref, ea_ref, dg_ref, WbT_ref, attl_ref, P_ref):
    Eb = _dot(ea_ref[...], WbT_ref[...])
    Ag = jnp.concatenate([Ag_ref[0], Ag_ref[1]], axis=1)
    m = _leaky(Ag + Eb)
    l = _leaky(jnp.sum(m * attl_ref[...], axis=1) + dg_ref[0, 0, :])
    w = jnp.exp(l)
    P = m * w[:, None]
    col = lax.broadcasted_iota(jnp.int32, (512, HP), 1)
    P = jnp.where(col == H, w[:, None], P)
    P_ref[0] = P[:, :HH]
    P_ref[1] = P[:, HH:]


def _tc_edge1(Ag2, ea, dg, WbT, attl):
    return pl.pallas_call(
        _tc_edge1_body,
        grid=(EBLK,),
        in_specs=[
            pl.BlockSpec((2, 512, HH), lambda i: (0, i, 0)),
            pl.BlockSpec((512, ED), lambda i: (i, 0)),
            pl.BlockSpec((1, 1, 512), lambda i: (i, 0, 0)),
            pl.BlockSpec((ED, HP), lambda i: (0, 0)),
            pl.BlockSpec((1, HP), lambda i: (0, 0)),
        ],
        out_specs=pl.BlockSpec((2, 512, HH), lambda i: (0, i, 0)),
        out_shape=jax.ShapeDtypeStruct((2, E, HH), jnp.float32),
    )(Ag2, ea, dg, WbT, attl)


def _tc_node_mid_body(S2_ref, h0_ref, W2T_ref, gb_ref, WihT_ref, WhhT_ref,
                      bih_ref, bhh_ref, WgT_ref, asrc_ref, adst_ref,
                      xc_ref, xt2_ref, sa_ref, sb_ref):
    M = jnp.concatenate([S2_ref[0], S2_ref[1]], axis=1)
    s = M[:, H]
    h = _elu(_dot(M, W2T_ref[...]) / (s[:, None] + 1e-16) + gb_ref[...])
    h0 = h0_ref[...]
    xc = jnp.maximum(_gru(h, h0, WihT_ref[...], WhhT_ref[...],
                          bih_ref[...], bhh_ref[...]), 0.0)
    xc_ref[...] = xc
    xt = _dot(xc, WgT_ref[...])
    xt2_ref[0] = xt[:, :HH]
    xt2_ref[1] = xt[:, HH:]
    sa_ref[0, 0, :] = jnp.sum(xt * asrc_ref[...], axis=1)
    sb_ref[0, 0, :] = jnp.sum(xt * adst_ref[...], axis=1)


def _tc_node_mid(S2, h0, W2T, gb, WihT, WhhT, bih, bhh, WgT, asrc, adst):
    return pl.pallas_call(
        _tc_node_mid_body,
        grid=(NBLK,),
        in_specs=[
            pl.BlockSpec((2, 256, HH), lambda i: (0, i, 0)),
            pl.BlockSpec((256, HP), lambda i: (i, 0)),
            pl.BlockSpec((HP, HP), lambda i: (0, 0)),
            pl.BlockSpec((1, HP), lambda i: (0, 0)),
            pl.BlockSpec((HP, H3P), lambda i: (0, 0)),
            pl.BlockSpec((HP, H3P), lambda i: (0, 0)),
            pl.BlockSpec((1, H3P), lambda i: (0, 0)),
            pl.BlockSpec((1, H3P), lambda i: (0, 0)),
            pl.BlockSpec((HP, HP), lambda i: (0, 0)),
            pl.BlockSpec((1, HP), lambda i: (0, 0)),
            pl.BlockSpec((1, HP), lambda i: (0, 0)),
        ],
        out_specs=[
            pl.BlockSpec((256, HP), lambda i: (i, 0)),
            pl.BlockSpec((2, 256, HH), lambda i: (0, i, 0)),
            pl.BlockSpec((1, 1, 256), lambda i: (i, 0, 0)),
            pl.BlockSpec((1, 1, 256), lambda i: (i, 0, 0)),
        ],
        out_shape=[
            jax.ShapeDtypeStruct((NP, HP), jnp.float32),
            jax.ShapeDtypeStruct((2, NP, HH), jnp.float32),
            jax.ShapeDtypeStruct((NBLK, 1, 256), jnp.float32),
            jax.ShapeDtypeStruct((NBLK, 1, 256), jnp.float32),
        ],
    )(S2, h0, W2T, gb, WihT, WhhT, bih, bhh, WgT, asrc, adst)


def _tc_edge2_body(Xg_ref, w_ref, P_ref):
    Xg = jnp.concatenate([Xg_ref[0], Xg_ref[1]], axis=1)
    w = w_ref[0, 0, :]
    P = Xg * w[:, None]
    col = lax.broadcasted_iota(jnp.int32, (512, HP), 1)
    P = jnp.where(col == H, w[:, None], P)
    P_ref[0] = P[:, :HH]
    P_ref[1] = P[:, HH:]


def _tc_edge2(Xg2, w):
    return pl.pallas_call(
        _tc_edge2_body,
        grid=(EBLK,),
        in_specs=[
            pl.BlockSpec((2, 512, HH), lambda i: (0, i, 0)),
            pl.BlockSpec((1, 1, 512), lambda i: (i, 0, 0)),
        ],
        out_specs=pl.BlockSpec((2, 512, HH), lambda i: (0, i, 0)),
        out_shape=jax.ShapeDtypeStruct((2, E, HH), jnp.float32),
    )(Xg2, w)


def _tc_node_post_body(X2_ref, xc_ref, gb_ref, WihT_ref, WhhT_ref, bih_ref,
                       bhh_ref, WmT_ref, asrc_ref, xc2_ref, xs_ref, an_ref):
    X = jnp.concatenate([X2_ref[0], X2_ref[1]], axis=1)
    s = X[:, H]
    h = _elu(X / (s[:, None] + 1e-16) + gb_ref[...])
    xc = xc_ref[...]
    xc2 = jnp.maximum(_gru(h, xc, WihT_ref[...], WhhT_ref[...],
                           bih_ref[...], bhh_ref[...]), 0.0)
    xc2_ref[...] = xc2
    xs = _dot(xc2, WmT_ref[...])
    xs_ref[...] = xs
    an_ref[0, 0, :] = jnp.sum(xs * asrc_ref[...], axis=1)


def _tc_node_post(X2, xc, gb, WihT, WhhT, bih, bhh, WmT, asrc):
    return pl.pallas_call(
        _tc_node_post_body,
        grid=(NBLK,),
        in_specs=[
            pl.BlockSpec((2, 256, HH), lambda i: (0, i, 0)),
            pl.BlockSpec((256, HP), lambda i: (i, 0)),
            pl.BlockSpec((1, HP), lambda i: (0, 0)),
            pl.BlockSpec((HP, H3P), lambda i: (0, 0)),
            pl.BlockSpec((HP, H3P), lambda i: (0, 0)),
            pl.BlockSpec((1, H3P), lambda i: (0, 0)),
            pl.BlockSpec((1, H3P), lambda i: (0, 0)),
            pl.BlockSpec((HP, HP), lambda i: (0, 0)),
            pl.BlockSpec((1, HP), lambda i: (0, 0)),
        ],
        out_specs=[
            pl.BlockSpec((256, HP), lambda i: (i, 0)),
            pl.BlockSpec((256, HP), lambda i: (i, 0)),
            pl.BlockSpec((1, 1, 256), lambda i: (i, 0, 0)),
        ],
        out_shape=[
            jax.ShapeDtypeStruct((NP, HP), jnp.float32),
            jax.ShapeDtypeStruct((NP, HP), jnp.float32),
            jax.ShapeDtypeStruct((NBLK, 1, 256), jnp.float32),
        ],
    )(X2, xc, gb, WihT, WhhT, bih, bhh, WmT, asrc)


def _tc_seg_sum_body(xc2_ref, b_ref, o_ref):
    i = pl.program_id(0)
    b = b_ref[0, 0, :]
    oh = (b[:, None] == lax.broadcasted_iota(jnp.int32, (256, G), 1)
          ).astype(jnp.float32)
    contrib = lax.dot_general(oh, xc2_ref[...], (((0,), (0,)), ((), ())),
                              preferred_element_type=jnp.float32)
    prev = jnp.where(i == 0, jnp.zeros_like(contrib), o_ref[...])
    acc = prev + contrib
    o_ref[...] = jnp.where(i == NBLK - 1, jnp.maximum(acc, 0.0), acc)


def _tc_seg_sum(xc2, batch3):
    return pl.pallas_call(
        _tc_seg_sum_body,
        grid=(NBLK,),
        in_specs=[
            pl.BlockSpec((256, HP), lambda i: (i, 0)),
            pl.BlockSpec((1, 1, 256), lambda i: (i, 0, 0)),
        ],
        out_specs=pl.BlockSpec((G, HP), lambda i: (0, 0)),
        out_shape=jax.ShapeDtypeStruct((G, HP), jnp.float32),
    )(xc2, batch3)


def _tc_mol_iter_body(out_ref, xs_ref, an_ref, b_ref, WmT_ref, adst_ref,
                      mb_ref, WihT_ref, WhhT_ref, bih_ref, bhh_ref,
                      onew_ref, dd_scr, s3_scr, Hm_scr):
    i = pl.program_id(0)

    @pl.when(i == 0)
    def _():
        od = _dot(out_ref[...], WmT_ref[...])
        dd_scr[0, :] = jnp.sum(od * adst_ref[...], axis=1)
        s3_scr[...] = jnp.zeros_like(s3_scr)
        Hm_scr[...] = jnp.zeros_like(Hm_scr)

    b = b_ref[0, 0, :]
    oh = (b[:, None] == lax.broadcasted_iota(jnp.int32, (256, G), 1)
          ).astype(jnp.float32)
    ddb = jnp.sum(oh * dd_scr[0, :][None, :], axis=1)
    l3 = _leaky(an_ref[0, 0, :] + ddb)
    e3 = jnp.exp(l3)
    s3_scr[0, :] += jnp.sum(oh * e3[:, None], axis=0)
    Hm_scr[...] += lax.dot_general(oh, xs_ref[...] * e3[:, None],
                                   (((0,), (0,)), ((), ())),
                                   preferred_element_type=jnp.float32)

    @pl.when(i == NBLK - 1)
    def _():
        s3 = s3_scr[0, :]
        h = _elu(Hm_scr[...] / (s3[:, None] + 1e-16) + mb_ref[...])
        o = out_ref[...]
        onew = _gru(h, o, WihT_ref[...], WhhT_ref[...],
                    bih_ref[...], bhh_ref[...])
        onew_ref[...] = jnp.maximum(onew, 0.0)


def _tc_mol_iter(out, xs, an, batch3, WmT, adst, mb, WihT, WhhT, bih, bhh):
    return pl.pallas_call(
        _tc_mol_iter_body,
        grid=(NBLK,),
        in_specs=[
            pl.BlockSpec((G, HP), lambda i: (0, 0)),
            pl.BlockSpec((256, HP), lambda i: (i, 0)),
            pl.BlockSpec((1, 1, 256), lambda i: (i, 0, 0)),
            pl.BlockSpec((1, 1, 256), lambda i: (i, 0, 0)),
            pl.BlockSpec((HP, HP), lambda i: (0, 0)),
            pl.BlockSpec((1, HP), lambda i: (0, 0)),
            pl.BlockSpec((1, HP), lambda i: (0, 0)),
            pl.BlockSpec((HP, H3P), lambda i: (0, 0)),
            pl.BlockSpec((HP, H3P), lambda i: (0, 0)),
            pl.BlockSpec((1, H3P), lambda i: (0, 0)),
            pl.BlockSpec((1, H3P), lambda i: (0, 0)),
        ],
        out_specs=pl.BlockSpec((G, HP), lambda i: (0, 0)),
        out_shape=jax.ShapeDtypeStruct((G, HP), jnp.float32),
        scratch_shapes=[
            pltpu.VMEM((1, G), jnp.float32),
            pltpu.VMEM((1, G), jnp.float32),
            pltpu.VMEM((G, HP), jnp.float32),
        ],
    )(out, xs, an, batch3, WmT, adst, mb, WihT, WhhT, bih, bhh)


def _tc_head_body(out_ref, W1T_ref, b1_ref, W2_ref, b2_ref, o_ref):
    h1 = jnp.maximum(_dot(out_ref[...], W1T_ref[...]) + b1_ref[...], 0.0)
    o_ref[...] = _dot(h1, W2_ref[...]) + b2_ref[...]


def _tc_head(out, W1T, b1, W2blk, b2):
    return pl.pallas_call(
        _tc_head_body,
        grid=(1,),
        in_specs=[
            pl.BlockSpec((G, HP), lambda i: (0, 0)),
            pl.BlockSpec((HP, 1280), lambda i: (0, 0)),
            pl.BlockSpec((1, 1280), lambda i: (0, 0)),
            pl.BlockSpec((1280, 128), lambda i: (0, 0)),
            pl.BlockSpec((1, 128), lambda i: (0, 0)),
        ],
        out_specs=pl.BlockSpec((G, 128), lambda i: (0, 0)),
        out_shape=jax.ShapeDtypeStruct((G, 128), jnp.float32),
    )(out, W1T, b1, W2blk, b2)


def _padT(W, rows, cols):
    r0, c0 = W.shape
    Wp = jnp.zeros((rows, cols), W.dtype).at[:r0, :c0].set(W)
    return Wp.T


def _padv(v, n):
    return jnp.zeros((1, n), v.dtype).at[0, :v.shape[0]].set(v)


def _pad_gru(Wih, Whh, bih, bhh):
    WihT = jnp.zeros((H3P, HP), Wih.dtype)
    WhhT = jnp.zeros((H3P, HP), Whh.dtype)
    bihp = jnp.zeros((1, H3P), bih.dtype)
    bhhp = jnp.zeros((1, H3P), bhh.dtype)
    for k in range(3):
        WihT = WihT.at[k * HP:k * HP + H, :H].set(Wih[k * H:(k + 1) * H])
        WhhT = WhhT.at[k * HP:k * HP + H, :H].set(Whh[k * H:(k + 1) * H])
        bihp = bihp.at[0, k * HP:k * HP + H].set(bih[k * H:(k + 1) * H])
        bhhp = bhhp.at[0, k * HP:k * HP + H].set(bhh[k * H:(k + 1) * H])
    return WihT.T, WhhT.T, bihp, bhhp


def kernel(x, edge_index, edge_attr, batch, lin1_W, lin1_b, gate_lin1_W,
           gate_lin2_W, gate_att_l, gate_att_r, gate_bias, gru0_Wih, gru0_Whh,
           gru0_bih, gru0_bhh, gat_W, gat_att_src, gat_att_dst, gat_bias,
           gru1_Wih, gru1_Whh, gru1_bih, gru1_bhh, mol_W, mol_att_src,
           mol_att_dst, mol_bias, mgru_Wih, mgru_Whh, mgru_bih, mgru_bhh,
           head_W1, head_b1, head_W2, head_b2):
    src = edge_index[0]
    dst = edge_index[1]
    xp = jnp.zeros((NP, FIN), jnp.float32).at[:N0].set(x)
    batchp = jnp.full((NP,), G, jnp.int32).at[:N0].set(batch)
    batch3 = batchp.reshape(NBLK, 1, 256)

    W1T = _padT(lin1_W, HP, FIN)
    b1 = _padv(lin1_b, HP)
    WaT = _padT(gate_lin1_W[:, :H], HP, HP)
    WbT = _padT(gate_lin1_W[:, H:], HP, ED)
    attl = _padv(gate_att_l, HP)
    attr_ = _padv(gate_att_r, HP)
    W2T = _padT(gate_lin2_W, HP, HP)
    gb = _padv(gate_bias, HP)
    g0 = _pad_gru(gru0_Wih, gru0_Whh, gru0_bih, gru0_bhh)
    g1 = _pad_gru(gru1_Wih, gru1_Whh, gru1_bih, gru1_bhh)
    gm = _pad_gru(mgru_Wih, mgru_Whh, mgru_bih, mgru_bhh)
    WgT = _padT(gat_W, HP, HP)
    gasrc = _padv(gat_att_src, HP)
    gadst = _padv(gat_att_dst, HP)
    gatb = _padv(gat_bias, HP)
    WmT = _padT(mol_W, HP, HP)
    masrc = _padv(mol_att_src, HP)
    madst = _padv(mol_att_dst, HP)
    mb = _padv(mol_bias, HP)
    W1r = head_W1.reshape(NT * (H // 2), H)
    hW1T = _padT(W1r, 1280, HP)
    hb1 = _padv(head_b1.reshape(-1), 1280)
    W2blk = jnp.zeros((1280, 128), jnp.float32)
    for k in range(NT):
        W2blk = W2blk.at[k * (H // 2):(k + 1) * (H // 2), k].set(head_W2[k, 0])
    hb2 = _padv(head_b2[:, 0], 128)

    zero_tile = jnp.zeros((NPT, HH), jnp.float32)

    h0, A2, d3 = _tc_node_pre(xp, W1T, b1, WaT, attr_)

    Ag2 = _sc_gather_rows(A2.reshape(2 * NP, HH), src)
    dg = _sc_gather_scal(d3.reshape(NP), dst)
    P2 = _tc_edge1(Ag2.reshape(2, E, HH), edge_attr,
                   dg.reshape(EBLK, 1, 512), WbT, attl)
    S2 = _sc_scatter_rows(P2.reshape(2 * E, HH), dst, zero_tile)

    xc, xt2, sa3, sb3 = _tc_node_mid(S2.reshape(2, NP, HH), h0, W2T, gb,
                                     *g0, WgT, gasrc, gadst)

    X2 = _sc_gat_fused(xt2.reshape(2 * NP, HH), src, dst,
                       sa3.reshape(NP), sb3.reshape(NP), zero_tile)

    xc2, xs, an3 = _tc_node_post(X2.reshape(2, NP, HH), xc, gatb,
                                 *g1, WmT, masrc)

    out = _tc_seg_sum(xc2, batch3)
    for _ in range(2):
        out = _tc_mol_iter(out, xs, an3, batch3, WmT, madst, mb, *gm)

    logits = _tc_head(out, hW1T, hb1, W2blk, hb2)
    return logits[:, :NT]


# speedup vs baseline: 10.7296x; 1.2986x over previous
import functools

import jax
import jax.numpy as jnp
from jax import lax
from jax.experimental import pallas as pl
from jax.experimental.pallas import tpu as pltpu
from jax.experimental.pallas import tpu_sc as plsc

N0 = 10000
NP = 10240
E = 320000
FIN = 128
H = 200
HP = 256
HH = 128
H3P = 768
ED = 16
G = 512
NT = 12
NBLK = NP // 256
EBLK = E // 512

NC = 2
NS = 16
C = 80
EPT = E // NS
EPW = E // (NC * NS)
NPT = NP // NS

@functools.cache
def _mesh():
    return plsc.VectorSubcoreMesh(core_axis_name="c", subcore_axis_name="s",
                                  num_cores=NC, num_subcores=NS)


def _leaky(v):
    return jnp.where(v > 0, v, 0.01 * v)


def _elu(v):
    return jnp.where(v > 0, v, jnp.exp(v) - 1.0)


@functools.cache
def _sc_gather1_k():
    NCH = EPT // C

    @functools.partial(
        pl.kernel,
        out_type=(jax.ShapeDtypeStruct((2 * E, HH), jnp.float32),
                  jax.ShapeDtypeStruct((E,), jnp.float32)),
        mesh=_mesh(),
        compiler_params=pltpu.CompilerParams(needs_layout_passes=False,
                                             use_tc_tiling_on_sc=False),
        scratch_types=[
            pltpu.VMEM((C,), jnp.int32),
            pltpu.VMEM((C,), jnp.int32),
            pltpu.VMEM((C,), jnp.int32),
            pltpu.VMEM((C,), jnp.int32),
            pltpu.VMEM((C, HH), jnp.float32),
            pltpu.VMEM((C, HH), jnp.float32),
            pltpu.VMEM((NP // 16, 16), jnp.float32),
            pltpu.VMEM((EPW,), jnp.int32),
            pltpu.VMEM((EPW,), jnp.float32),
            pltpu.SemaphoreType.DMA,
            pltpu.SemaphoreType.DMA,
            pltpu.SemaphoreType.DMA,
            pltpu.SemaphoreType.DMA,
            pltpu.SemaphoreType.DMA,
            pltpu.SemaphoreType.DMA,
        ],
    )
    def k(tab_hbm, src_hbm, dst_hbm, d_hbm, out_hbm, dg_hbm,
          sraw0, sraw1, sib0, sib1, rb0, rb1, dvm, dslab, ogslab,
          isem0, isem1, gsem0, gsem1, wsem0, wsem1):
        c = lax.axis_index("c")
        s = lax.axis_index("s")
        t0 = s * EPT
        off = c * NP
        w = s * NC + c
        w0 = w * EPW

        pltpu.sync_copy(d_hbm, dvm)
        pltpu.sync_copy(dst_hbm.at[pl.ds(w0, EPW)], dslab)

        def dg_body(j, carry):
            vd = dslab[pl.ds(j * 16, 16)]
            ogslab[pl.ds(j * 16, 16)] = plsc.load_gather(
                dvm, [lax.shift_right_logical(vd, 4), vd & 15])
            return carry

        lax.fori_loop(0, EPW // 16, dg_body, 0)
        pltpu.sync_copy(ogslab, dg_hbm.at[pl.ds(w0, EPW)])

        def start_idx(g, sraw, isem):
            pltpu.async_copy(src_hbm.at[pl.ds(t0 + g * C, C)], sraw, isem)

        def wait_idx(g, sraw, isem):
            pltpu.make_async_copy(
                src_hbm.at[pl.ds(t0 + g * C, C)], sraw, isem).wait()

        def fill_sib(sraw, sib):
            for j in range(C // 16):
                sib[pl.ds(j * 16, 16)] = sraw[pl.ds(j * 16, 16)] + off

        def start_gather(sib, rb, gsem):
            pltpu.async_copy(tab_hbm.at[sib], rb, gsem)

        def wait_gather(sib, rb, gsem):
            pltpu.make_async_copy(tab_hbm.at[sib], rb, gsem).wait()

        def start_wb(g, rb, wsem):
            pltpu.async_copy(rb, out_hbm.at[pl.ds(c * E + t0 + g * C, C)], wsem)

        def wait_wb(g, rb, wsem):
            pltpu.make_async_copy(
                rb, out_hbm.at[pl.ds(c * E + t0 + g * C, C)], wsem).wait()

        start_idx(0, sraw0, isem0)
        start_idx(1, sraw1, isem1)
        wait_idx(0, sraw0, isem0)
        fill_sib(sraw0, sib0)
        start_gather(sib0, rb0, gsem0)
        wait_idx(1, sraw1, isem1)
        fill_sib(sraw1, sib1)
        start_gather(sib1, rb1, gsem1)
        wait_gather(sib0, rb0, gsem0)
        start_wb(0, rb0, wsem0)
        start_idx(2, sraw0, isem0)
        wait_idx(2, sraw0, isem0)
        fill_sib(sraw0, sib0)
        wait_wb(0, rb0, wsem0)
        start_gather(sib0, rb0, gsem0)
        wait_gather(sib1, rb1, gsem1)
        start_wb(1, rb1, wsem1)
        start_idx(3, sraw1, isem1)

        def pair(p, carry):
            g = 2 * p
            wait_idx(g + 1, sraw1, isem1)
            fill_sib(sraw1, sib1)
            wait_wb(g - 1, rb1, wsem1)
            start_gather(sib1, rb1, gsem1)
            wait_gather(sib0, rb0, gsem0)
            start_wb(g, rb0, wsem0)
            start_idx(g + 2, sraw0, isem0)

            wait_idx(g + 2, sraw0, isem0)
            fill_sib(sraw0, sib0)
            wait_wb(g, rb0, wsem0)
            start_gather(sib0, rb0, gsem0)
            wait_gather(sib1, rb1, gsem1)
            start_wb(g + 1, rb1, wsem1)
            start_idx(g + 3, sraw1, isem1)
            return carry

        lax.fori_loop(1, NCH // 2 - 1, pair, 0)

        g = NCH - 2
        wait_idx(g + 1, sraw1, isem1)
        fill_sib(sraw1, sib1)
        wait_wb(g - 1, rb1, wsem1)
        start_gather(sib1, rb1, gsem1)
        wait_gather(sib0, rb0, gsem0)
        start_wb(g, rb0, wsem0)
        wait_wb(g, rb0, wsem0)
        wait_gather(sib1, rb1, gsem1)
        start_wb(g + 1, rb1, wsem1)
        wait_wb(g + 1, rb1, wsem1)

    return k


def _sc_gather1(tabf, src, dst, d):
    return _sc_gather1_k()(tabf, src, dst, d.reshape(NP // 16, 16))


@functools.cache
def _sc_scatter_rows_k():
    NCH = EPT // C

    @functools.partial(
        pl.kernel,
        out_type=jax.ShapeDtypeStruct((2 * NP, HH), jnp.float32),
        mesh=_mesh(),
        compiler_params=pltpu.CompilerParams(needs_layout_passes=False,
                                             use_tc_tiling_on_sc=False),
        scratch_types=[
            pltpu.VMEM((C,), jnp.int32),
            pltpu.VMEM((C,), jnp.int32),
            pltpu.VMEM((C,), jnp.int32),
            pltpu.VMEM((C,), jnp.int32),
            pltpu.VMEM((C, HH), jnp.float32),
            pltpu.VMEM((C, HH), jnp.float32),
            pltpu.VMEM_SHARED((NP, HH), jnp.float32),
            pltpu.SemaphoreType.DMA,
            pltpu.SemaphoreType.DMA,
            pltpu.SemaphoreType.DMA,
            pltpu.SemaphoreType.DMA,
            pltpu.SemaphoreType.DMA,
            pltpu.SemaphoreType.DMA,
        ],
    )
    def k(rows_hbm, idx_hbm, zero_hbm, out_hbm,
          draw0, draw1, dib0, dib1, rb0, rb1, acc,
          isem0, isem1, rsem0, rsem1, ssem0, ssem1):
        c = lax.axis_index("c")
        s = lax.axis_index("s")
        t0 = s * EPT
        pltpu.sync_copy(zero_hbm, acc.at[pl.ds(s * NPT, NPT)])
        plsc.subcore_barrier()

        def start_idx(g, draw, isem):
            pltpu.async_copy(idx_hbm.at[pl.ds(t0 + g * C, C)], draw, isem)

        def wait_idx(g, draw, isem):
            pltpu.make_async_copy(
                idx_hbm.at[pl.ds(t0 + g * C, C)], draw, isem).wait()

        def start_read(g, rb, rsem):
            pltpu.async_copy(
                rows_hbm.at[pl.ds(c * E + t0 + g * C, C)], rb, rsem)

        def wait_read(g, rb, rsem):
            pltpu.make_async_copy(
                rows_hbm.at[pl.ds(c * E + t0 + g * C, C)], rb, rsem).wait()

        def snap_dib(draw, dib):
            for j in range(C // 16):
                dib[pl.ds(j * 16, 16)] = draw[pl.ds(j * 16, 16)]

        def start_scatter(rb, dib, ssem):
            pltpu.async_copy(rb, acc.at[dib], ssem, add=True)

        def wait_scatter(rb, dib, ssem):
            pltpu.make_async_copy(rb, acc.at[dib], ssem).wait()

        start_idx(0, draw0, isem0)
        start_idx(1, draw1, isem1)
        start_read(0, rb0, rsem0)
        start_read(1, rb1, rsem1)
        wait_read(0, rb0, rsem0)
        wait_idx(0, draw0, isem0)
        snap_dib(draw0, dib0)
        start_scatter(rb0, dib0, ssem0)
        start_idx(2, draw0, isem0)
        wait_read(1, rb1, rsem1)
        wait_idx(1, draw1, isem1)
        snap_dib(draw1, dib1)
        start_scatter(rb1, dib1, ssem1)
        start_idx(3, draw1, isem1)
        wait_scatter(rb0, dib0, ssem0)
        start_read(2, rb0, rsem0)

        def pair(p, carry):
            g = 2 * p
            wait_read(g, rb0, rsem0)
            wait_idx(g, draw0, isem0)
            snap_dib(draw0, dib0)
            start_scatter(rb0, dib0, ssem0)
            start_idx(g + 2, draw0, isem0)
            wait_scatter(rb1, dib1, ssem1)
            start_read(g + 1, rb1, rsem1)

            wait_read(g + 1, rb1, rsem1)
            wait_idx(g + 1, draw1, isem1)
            snap_dib(draw1, dib1)
            start_scatter(rb1, dib1, ssem1)
            start_idx(g + 3, draw1, isem1)
            wait_scatter(rb0, dib0, ssem0)
            start_read(g + 2, rb0, rsem0)
            return carry

        lax.fori_loop(1, NCH // 2 - 1, pair, 0)

        g = NCH - 2
        wait_read(g, rb0, rsem0)
        wait_idx(g, draw0, isem0)
        snap_dib(draw0, dib0)
        start_scatter(rb0, dib0, ssem0)
        wait_scatter(rb1, dib1, ssem1)
        start_read(g + 1, rb1, rsem1)
        wait_read(g + 1, rb1, rsem1)
        wait_idx(g + 1, draw1, isem1)
        snap_dib(draw1, dib1)
        start_scatter(rb1, dib1, ssem1)
        wait_scatter(rb0, dib0, ssem0)
        wait_scatter(rb1, dib1, ssem1)

        plsc.subcore_barrier()
        pltpu.sync_copy(acc.at[pl.ds(s * NPT, NPT)],
                        out_hbm.at[pl.ds(c * NP + s * NPT, NPT)])

    return k


def _sc_scatter_rows(rowsf, idx, zero_tile):
    return _sc_scatter_rows_k()(rowsf, idx, zero_tile)


@functools.cache
def _sc_gather_scal_k():
    @functools.partial(
        pl.kernel,
        out_type=jax.ShapeDtypeStruct((E,), jnp.float32),
        mesh=_mesh(),
        compiler_params=pltpu.CompilerParams(needs_layout_passes=False, use_tc_tiling_on_sc=False),
        scratch_types=[
            pltpu.VMEM((NP // 16, 16), jnp.float32),
            pltpu.VMEM((C,), jnp.int32),
            pltpu.VMEM((C,), jnp.float32),
        ],
    )
    def k(tab_hbm, idx_hbm, out_hbm, tv, ibuf, obuf):
        c = lax.axis_index("c")
        s = lax.axis_index("s")
        w = s * NC + c
        pltpu.sync_copy(tab_hbm, tv)
        t0 = w * EPW

        def body(g, carry):
            e0 = t0 + g * C
            pltpu.sync_copy(idx_hbm.at[pl.ds(e0, C)], ibuf)
            for j in range(C // 16):
                v = ibuf[pl.ds(j * 16, 16)]
                obuf[pl.ds(j * 16, 16)] = plsc.load_gather(
                    tv, [lax.shift_right_logical(v, 4), v & 15])
            pltpu.sync_copy(obuf, out_hbm.at[pl.ds(e0, C)])
            return carry

        lax.fori_loop(0, EPW // C, body, 0)

    return k


def _sc_gather_scal(tab, idx):
    return _sc_gather_scal_k()(tab.reshape(NP // 16, 16), idx)


@functools.cache
def _sc_edge_w_k():
    @functools.partial(
        pl.kernel,
        out_type=jax.ShapeDtypeStruct((E,), jnp.float32),
        mesh=_mesh(),
        compiler_params=pltpu.CompilerParams(needs_layout_passes=False, use_tc_tiling_on_sc=False),
        scratch_types=[
            pltpu.VMEM((NP // 16, 16), jnp.float32),
            pltpu.VMEM((NP // 16, 16), jnp.float32),
            pltpu.VMEM((C,), jnp.int32),
            pltpu.VMEM((C,), jnp.int32),
            pltpu.VMEM((C,), jnp.float32),
        ],
    )
    def k(sa_hbm, sb_hbm, src_hbm, dst_hbm, out_hbm, av, bv, sbuf, dbuf, obuf):
        c = lax.axis_index("c")
        s = lax.axis_index("s")
        w = s * NC + c
        pltpu.sync_copy(sa_hbm, av)
        pltpu.sync_copy(sb_hbm, bv)
        t0 = w * EPW

        def body(g, carry):
            e0 = t0 + g * C
            pltpu.sync_copy(src_hbm.at[pl.ds(e0, C)], sbuf)
            pltpu.sync_copy(dst_hbm.at[pl.ds(e0, C)], dbuf)
            for j in range(C // 16):
                vs = sbuf[pl.ds(j * 16, 16)]
                vd = dbuf[pl.ds(j * 16, 16)]
                va = plsc.load_gather(
                    av, [lax.shift_right_logical(vs, 4), vs & 15])
                vb = plsc.load_gather(
                    bv, [lax.shift_right_logical(vd, 4), vd & 15])
                x = va + vb
                l = jnp.where(x > 0, x, 0.01 * x)
                obuf[pl.ds(j * 16, 16)] = jnp.exp(l)
            pltpu.sync_copy(obuf, out_hbm.at[pl.ds(e0, C)])
            return carry

        lax.fori_loop(0, EPW // C, body, 0)

    return k


def _sc_edge_w(sa, sb, src, dst):
    return _sc_edge_w_k()(sa.reshape(NP // 16, 16), sb.reshape(NP // 16, 16),
                          src, dst)


@functools.cache
def _sc_gat_fused_k():
    NCH = EPT // C

    @functools.partial(
        pl.kernel,
        out_type=jax.ShapeDtypeStruct((2 * NP, HH), jnp.float32),
        mesh=_mesh(),
        compiler_params=pltpu.CompilerParams(needs_layout_passes=False,
                                             use_tc_tiling_on_sc=False),
        scratch_types=[
            pltpu.VMEM((NP // 16, 16), jnp.float32),
            pltpu.VMEM((NP // 16, 16), jnp.float32),
            pltpu.VMEM((C,), jnp.int32),
            pltpu.VMEM((C,), jnp.int32),
            pltpu.VMEM((C,), jnp.int32),
            pltpu.VMEM((C,), jnp.int32),
            pltpu.VMEM((C,), jnp.int32),
            pltpu.VMEM((C,), jnp.int32),
            pltpu.VMEM((C,), jnp.int32),
            pltpu.VMEM((C,), jnp.int32),
            pltpu.VMEM((C, HH), jnp.float32),
            pltpu.VMEM((C, HH), jnp.float32),
            pltpu.VMEM((C // 16, 16), jnp.float32),
            pltpu.VMEM_SHARED((NP, HH), jnp.float32),
            pltpu.SemaphoreType.DMA,
            pltpu.SemaphoreType.DMA,
            pltpu.SemaphoreType.DMA,
            pltpu.SemaphoreType.DMA,
            pltpu.SemaphoreType.DMA,
            pltpu.SemaphoreType.DMA,
        ],
    )
    def k(tab_hbm, src_hbm, dst_hbm, sa_hbm, sb_hbm, zero_hbm, out_hbm,
          savm, sbvm, sraw0, sraw1, draw0, draw1, sib0, sib1, dib0, dib1,
          rb0, rb1, obuf, acc, isem0, isem1, gsem0, gsem1, ssem0, ssem1):
        c = lax.axis_index("c")
        s = lax.axis_index("s")
        t0 = s * EPT
        off = c * NP
        lane = lax.iota(jnp.int32, 16)
        is_hi = lax.broadcast(c, (16,)) == 1
        wcol_mask = is_hi & (lane == (H - HH - 64))

        pltpu.sync_copy(sa_hbm, savm)
        pltpu.sync_copy(sb_hbm, sbvm)
        pltpu.sync_copy(zero_hbm, acc.at[pl.ds(s * NPT, NPT)])
        plsc.subcore_barrier()

        def start_idx(g, sraw, draw, isem):
            e0 = t0 + g * C
            pltpu.async_copy(src_hbm.at[pl.ds(e0, C)], sraw, isem)
            pltpu.async_copy(dst_hbm.at[pl.ds(e0, C)], draw, isem)

        def wait_idx(g, sraw, draw, isem):
            e0 = t0 + g * C
            pltpu.make_async_copy(src_hbm.at[pl.ds(e0, C)], sraw, isem).wait()
            pltpu.make_async_copy(dst_hbm.at[pl.ds(e0, C)], draw, isem).wait()

        def fill_sib(sraw, sib):
            for j in range(C // 16):
                sib[pl.ds(j * 16, 16)] = sraw[pl.ds(j * 16, 16)] + off

        def start_gather(sib, rb, gsem):
            pltpu.async_copy(tab_hbm.at[sib], rb, gsem)

        def wait_gather(sib, rb, gsem):
            pltpu.make_async_copy(tab_hbm.at[sib], rb, gsem).wait()

        def start_scatter(rb, dib, ssem):
            pltpu.async_copy(rb, acc.at[dib], ssem, add=True)

        def wait_scatter(rb, dib, ssem):
            pltpu.make_async_copy(rb, acc.at[dib], ssem).wait()

        def process(sraw, draw, dib, rb):
            for j in range(C // 16):
                vs = sraw[pl.ds(j * 16, 16)]
                vd = draw[pl.ds(j * 16, 16)]
                dib[pl.ds(j * 16, 16)] = vd
                va = plsc.load_gather(
                    savm, [lax.shift_right_logical(vs, 4), vs & 15])
                vb = plsc.load_gather(
                    sbvm, [lax.shift_right_logical(vd, 4), vd & 15])
                x = va + vb
                obuf[j] = jnp.exp(jnp.where(x > 0, x, 0.01 * x))

            def row_body(r, carry):
                hi = lax.broadcast(lax.shift_right_logical(r, 4), (16,))
                lo = lax.broadcast(r & 15, (16,))
                wspl = plsc.load_gather(obuf, [hi, lo])
                for kk in range(HH // 16):
                    v = rb[r, pl.ds(kk * 16, 16)] * wspl
                    if kk == 4:
                        v = jnp.where(wcol_mask, wspl, v)
                    rb[r, pl.ds(kk * 16, 16)] = v
                return carry

            lax.fori_loop(0, C, row_body, 0)

        start_idx(0, sraw0, draw0, isem0)
        start_idx(1, sraw1, draw1, isem1)
        wait_idx(0, sraw0, draw0, isem0)
        fill_sib(sraw0, sib0)
        start_gather(sib0, rb0, gsem0)
        wait_idx(1, sraw1, draw1, isem1)
        fill_sib(sraw1, sib1)
        start_gather(sib1, rb1, gsem1)
        wait_gather(sib0, rb0, gsem0)
        process(sraw0, draw0, dib0, rb0)
        start_scatter(rb0, dib0, ssem0)
        start_idx(2, sraw0, draw0, isem0)
        wait_idx(2, sraw0, draw0, isem0)
        fill_sib(sraw0, sib0)
        wait_scatter(rb0, dib0, ssem0)
        start_gather(sib0, rb0, gsem0)
        wait_gather(sib1, rb1, gsem1)
        process(sraw1, draw1, dib1, rb1)
        start_scatter(rb1, dib1, ssem1)
        start_idx(3, sraw1, draw1, isem1)

        def pair(p, carry):
            g = 2 * p
            wait_idx(g + 1, sraw1, draw1, isem1)
            fill_sib(sraw1, sib1)
            wait_scatter(rb1, dib1, ssem1)
            start_gather(sib1, rb1, gsem1)
            wait_gather(sib0, rb0, gsem0)
            process(sraw0, draw0, dib0, rb0)
            start_scatter(rb0, dib0, ssem0)
            start_idx(g + 2, sraw0, draw0, isem0)
            wait_idx(g + 2, sraw0, draw0, isem0)
            fill_sib(sraw0, sib0)
            wait_scatter(rb0, dib0, ssem0)
            start_gather(sib0, rb0, gsem0)
            wait_gather(sib1, rb1, gsem1)
            process(sraw1, draw1, dib1, rb1)
            start_scatter(rb1, dib1, ssem1)
            start_idx(g + 3, sraw1, draw1, isem1)
            return carry

        lax.fori_loop(1, NCH // 2 - 1, pair, 0)

        wait_idx(NCH - 1, sraw1, draw1, isem1)
        fill_sib(sraw1, sib1)
        wait_scatter(rb1, dib1, ssem1)
        start_gather(sib1, rb1, gsem1)
        wait_gather(sib0, rb0, gsem0)
        process(sraw0, draw0, dib0, rb0)
        start_scatter(rb0, dib0, ssem0)
        wait_scatter(rb0, dib0, ssem0)
        wait_gather(sib1, rb1, gsem1)
        process(sraw1, draw1, dib1, rb1)
        start_scatter(rb1, dib1, ssem1)
        wait_scatter(rb1, dib1, ssem1)

        plsc.subcore_barrier()
        pltpu.sync_copy(acc.at[pl.ds(s * NPT, NPT)],
                        out_hbm.at[pl.ds(c * NP + s * NPT, NPT)])

    return k


def _sc_gat_fused(tabf, src, dst, sa, sb, zero_tile):
    return _sc_gat_fused_k()(tabf, src, dst, sa.reshape(NP // 16, 16),
                             sb.reshape(NP // 16, 16), zero_tile)


def _dot(a, b):
    return jnp.dot(a, b, preferred_element_type=jnp.float32)


def _gru(x, h, WihT, WhhT, bih, bhh):
    gi = _dot(x, WihT) + bih
    gh = _dot(h, WhhT) + bhh
    i_r, i_z, i_n = gi[:, :HP], gi[:, HP:2 * HP], gi[:, 2 * HP:]
    h_r, h_z, h_n = gh[:, :HP], gh[:, HP:2 * HP], gh[:, 2 * HP:]
    r = jax.nn.sigmoid(i_r + h_r)
    z = jax.nn.sigmoid(i_z + h_z)
    n = jnp.tanh(i_n + r * h_n)
    return (1.0 - z) * n + z * h


def _tc_node_pre_body(x_ref, W1T_ref, b1_ref, WaT_ref, attr_ref,
                      h0_ref, A2_ref, d_ref):
    x = x_ref[...]
    h0 = _leaky(_dot(x, W1T_ref[...]) + b1_ref[...])
    h0_ref[...] = h0
    A = _dot(h0, WaT_ref[...])
    A2_ref[0] = A[:, :HH]
    A2_ref[1] = A[:, HH:]
    d_ref[0, 0, :] = jnp.sum(h0 * attr_ref[...], axis=1)


def _tc_node_pre(x, W1T, b1, WaT, attr):
    return pl.pallas_call(
        _tc_node_pre_body,
        grid=(NBLK,),
        in_specs=[
            pl.BlockSpec((256, FIN), lambda i: (i, 0)),
            pl.BlockSpec((FIN, HP), lambda i: (0, 0)),
            pl.BlockSpec((1, HP), lambda i: (0, 0)),
            pl.BlockSpec((HP, HP), lambda i: (0, 0)),
            pl.BlockSpec((1, HP), lambda i: (0, 0)),
        ],
        out_specs=[
            pl.BlockSpec((256, HP), lambda i: (i, 0)),
            pl.BlockSpec((2, 256, HH), lambda i: (0, i, 0)),
            pl.BlockSpec((1, 1, 256), lambda i: (i, 0, 0)),
        ],
        out_shape=[
            jax.ShapeDtypeStruct((NP, HP), jnp.float32),
            jax.ShapeDtypeStruct((2, NP, HH), jnp.float32),
            jax.ShapeDtypeStruct((NBLK, 1, 256), jnp.float32),
        ],
    )(x, W1T, b1, WaT, attr)


def _tc_edge1_body(Ag_ref, ea_ref, dg_ref, WbT_ref, attl_ref, P_ref):
    Eb = _dot(ea_ref[...], WbT_ref[...])
    Ag = jnp.concatenate([Ag_ref[0], Ag_ref[1]], axis=1)
    m = _leaky(Ag + Eb)
    l = _leaky(jnp.sum(m * attl_ref[...], axis=1) + dg_ref[0, 0, :])
    w = jnp.exp(l)
    P = m * w[:, None]
    col = lax.broadcasted_iota(jnp.int32, (512, HP), 1)
    P = jnp.where(col == H, w[:, None], P)
    P_ref[0] = P[:, :HH]
    P_ref[1] = P[:, HH:]


def _tc_edge1(Ag2, ea, dg, WbT, attl):
    return pl.pallas_call(
        _tc_edge1_body,
        grid=(EBLK,),
        in_specs=[
            pl.BlockSpec((2, 512, HH), lambda i: (0, i, 0)),
            pl.BlockSpec((512, ED), lambda i: (i, 0)),
            pl.BlockSpec((1, 1, 512), lambda i: (i, 0, 0)),
            pl.BlockSpec((ED, HP), lambda i: (0, 0)),
            pl.BlockSpec((1, HP), lambda i: (0, 0)),
        ],
        out_specs=pl.BlockSpec((2, 512, HH), lambda i: (0, i, 0)),
        out_shape=jax.ShapeDtypeStruct((2, E, HH), jnp.float32),
    )(Ag2, ea, dg, WbT, attl)


def _tc_node_mid_body(S2_ref, h0_ref, W2T_ref, gb_ref, WihT_ref, WhhT_ref,
                      bih_ref, bhh_ref, WgT_ref, asrc_ref, adst_ref,
                      xc_ref, xt2_ref, sa_ref, sb_ref):
    M = jnp.concatenate([S2_ref[0], S2_ref[1]], axis=1)
    s = M[:, H]
    h = _elu(_dot(M, W2T_ref[...]) / (s[:, None] + 1e-16) + gb_ref[...])
    h0 = h0_ref[...]
    xc = jnp.maximum(_gru(h, h0, WihT_ref[...], WhhT_ref[...],
                          bih_ref[...], bhh_ref[...]), 0.0)
    xc_ref[...] = xc
    xt = _dot(xc, WgT_ref[...])
    xt2_ref[0] = xt[:, :HH]
    xt2_ref[1] = xt[:, HH:]
    sa_ref[0, 0, :] = jnp.sum(xt * asrc_ref[...], axis=1)
    sb_ref[0, 0, :] = jnp.sum(xt * adst_ref[...], axis=1)


def _tc_node_mid(S2, h0, W2T, gb, WihT, WhhT, bih, bhh, WgT, asrc, adst):
    return pl.pallas_call(
        _tc_node_mid_body,
        grid=(NBLK,),
        in_specs=[
            pl.BlockSpec((2, 256, HH), lambda i: (0, i, 0)),
            pl.BlockSpec((256, HP), lambda i: (i, 0)),
            pl.BlockSpec((HP, HP), lambda i: (0, 0)),
            pl.BlockSpec((1, HP), lambda i: (0, 0)),
            pl.BlockSpec((HP, H3P), lambda i: (0, 0)),
            pl.BlockSpec((HP, H3P), lambda i: (0, 0)),
            pl.BlockSpec((1, H3P), lambda i: (0, 0)),
            pl.BlockSpec((1, H3P), lambda i: (0, 0)),
            pl.BlockSpec((HP, HP), lambda i: (0, 0)),
            pl.BlockSpec((1, HP), lambda i: (0, 0)),
            pl.BlockSpec((1, HP), lambda i: (0, 0)),
        ],
        out_specs=[
            pl.BlockSpec((256, HP), lambda i: (i, 0)),
            pl.BlockSpec((2, 256, HH), lambda i: (0, i, 0)),
            pl.BlockSpec((1, 1, 256), lambda i: (i, 0, 0)),
            pl.BlockSpec((1, 1, 256), lambda i: (i, 0, 0)),
        ],
        out_shape=[
            jax.ShapeDtypeStruct((NP, HP), jnp.float32),
            jax.ShapeDtypeStruct((2, NP, HH), jnp.float32),
            jax.ShapeDtypeStruct((NBLK, 1, 256), jnp.float32),
            jax.ShapeDtypeStruct((NBLK, 1, 256), jnp.float32),
        ],
    )(S2, h0, W2T, gb, WihT, WhhT, bih, bhh, WgT, asrc, adst)


def _tc_edge2_body(Xg_ref, w_ref, P_ref):
    Xg = jnp.concatenate([Xg_ref[0], Xg_ref[1]], axis=1)
    w = w_ref[0, 0, :]
    P = Xg * w[:, None]
    col = lax.broadcasted_iota(jnp.int32, (512, HP), 1)
    P = jnp.where(col == H, w[:, None], P)
    P_ref[0] = P[:, :HH]
    P_ref[1] = P[:, HH:]


def _tc_edge2(Xg2, w):
    return pl.pallas_call(
        _tc_edge2_body,
        grid=(EBLK,),
        in_specs=[
            pl.BlockSpec((2, 512, HH), lambda i: (0, i, 0)),
            pl.BlockSpec((1, 1, 512), lambda i: (i, 0, 0)),
        ],
        out_specs=pl.BlockSpec((2, 512, HH), lambda i: (0, i, 0)),
        out_shape=jax.ShapeDtypeStruct((2, E, HH), jnp.float32),
    )(Xg2, w)


def _tc_node_post_body(X2_ref, xc_ref, gb_ref, WihT_ref, WhhT_ref, bih_ref,
                       bhh_ref, WmT_ref, asrc_ref, xc2_ref, xs_ref, an_ref):
    X = jnp.concatenate([X2_ref[0], X2_ref[1]], axis=1)
    s = X[:, H]
    h = _elu(X / (s[:, None] + 1e-16) + gb_ref[...])
    xc = xc_ref[...]
    xc2 = jnp.maximum(_gru(h, xc, WihT_ref[...], WhhT_ref[...],
                           bih_ref[...], bhh_ref[...]), 0.0)
    xc2_ref[...] = xc2
    xs = _dot(xc2, WmT_ref[...])
    xs_ref[...] = xs
    an_ref[0, 0, :] = jnp.sum(xs * asrc_ref[...], axis=1)


def _tc_node_post(X2, xc, gb, WihT, WhhT, bih, bhh, WmT, asrc):
    return pl.pallas_call(
        _tc_node_post_body,
        grid=(NBLK,),
        in_specs=[
            pl.BlockSpec((2, 256, HH), lambda i: (0, i, 0)),
            pl.BlockSpec((256, HP), lambda i: (i, 0)),
            pl.BlockSpec((1, HP), lambda i: (0, 0)),
            pl.BlockSpec((HP, H3P), lambda i: (0, 0)),
            pl.BlockSpec((HP, H3P), lambda i: (0, 0)),
            pl.BlockSpec((1, H3P), lambda i: (0, 0)),
            pl.BlockSpec((1, H3P), lambda i: (0, 0)),
            pl.BlockSpec((HP, HP), lambda i: (0, 0)),
            pl.BlockSpec((1, HP), lambda i: (0, 0)),
        ],
        out_specs=[
            pl.BlockSpec((256, HP), lambda i: (i, 0)),
            pl.BlockSpec((256, HP), lambda i: (i, 0)),
            pl.BlockSpec((1, 1, 256), lambda i: (i, 0, 0)),
        ],
        out_shape=[
            jax.ShapeDtypeStruct((NP, HP), jnp.float32),
            jax.ShapeDtypeStruct((NP, HP), jnp.float32),
            jax.ShapeDtypeStruct((NBLK, 1, 256), jnp.float32),
        ],
    )(X2, xc, gb, WihT, WhhT, bih, bhh, WmT, asrc)


def _tc_seg_sum_body(xc2_ref, b_ref, o_ref):
    i = pl.program_id(0)
    b = b_ref[0, 0, :]
    oh = (b[:, None] == lax.broadcasted_iota(jnp.int32, (256, G), 1)
          ).astype(jnp.float32)
    contrib = lax.dot_general(oh, xc2_ref[...], (((0,), (0,)), ((), ())),
                              preferred_element_type=jnp.float32)
    prev = jnp.where(i == 0, jnp.zeros_like(contrib), o_ref[...])
    acc = prev + contrib
    o_ref[...] = jnp.where(i == NBLK - 1, jnp.maximum(acc, 0.0), acc)


def _tc_seg_sum(xc2, batch3):
    return pl.pallas_call(
        _tc_seg_sum_body,
        grid=(NBLK,),
        in_specs=[
            pl.BlockSpec((256, HP), lambda i: (i, 0)),
            pl.BlockSpec((1, 1, 256), lambda i: (i, 0, 0)),
        ],
        out_specs=pl.BlockSpec((G, HP), lambda i: (0, 0)),
        out_shape=jax.ShapeDtypeStruct((G, HP), jnp.float32),
    )(xc2, batch3)


def _tc_mol_iter_body(out_ref, xs_ref, an_ref, b_ref, WmT_ref, adst_ref,
                      mb_ref, WihT_ref, WhhT_ref, bih_ref, bhh_ref,
                      onew_ref, dd_scr, s3_scr, Hm_scr):
    i = pl.program_id(0)

    @pl.when(i == 0)
    def _():
        od = _dot(out_ref[...], WmT_ref[...])
        dd_scr[0, :] = jnp.sum(od * adst_ref[...], axis=1)
        s3_scr[...] = jnp.zeros_like(s3_scr)
        Hm_scr[...] = jnp.zeros_like(Hm_scr)

    b = b_ref[0, 0, :]
    oh = (b[:, None] == lax.broadcasted_iota(jnp.int32, (256, G), 1)
          ).astype(jnp.float32)
    ddb = jnp.sum(oh * dd_scr[0, :][None, :], axis=1)
    l3 = _leaky(an_ref[0, 0, :] + ddb)
    e3 = jnp.exp(l3)
    s3_scr[0, :] += jnp.sum(oh * e3[:, None], axis=0)
    Hm_scr[...] += lax.dot_general(oh, xs_ref[...] * e3[:, None],
                                   (((0,), (0,)), ((), ())),
                                   preferred_element_type=jnp.float32)

    @pl.when(i == NBLK - 1)
    def _():
        s3 = s3_scr[0, :]
        h = _elu(Hm_scr[...] / (s3[:, None] + 1e-16) + mb_ref[...])
        o = out_ref[...]
        onew = _gru(h, o, WihT_ref[...], WhhT_ref[...],
                    bih_ref[...], bhh_ref[...])
        onew_ref[...] = jnp.maximum(onew, 0.0)


def _tc_mol_iter(out, xs, an, batch3, WmT, adst, mb, WihT, WhhT, bih, bhh):
    return pl.pallas_call(
        _tc_mol_iter_body,
        grid=(NBLK,),
        in_specs=[
            pl.BlockSpec((G, HP), lambda i: (0, 0)),
            pl.BlockSpec((256, HP), lambda i: (i, 0)),
            pl.BlockSpec((1, 1, 256), lambda i: (i, 0, 0)),
            pl.BlockSpec((1, 1, 256), lambda i: (i, 0, 0)),
            pl.BlockSpec((HP, HP), lambda i: (0, 0)),
            pl.BlockSpec((1, HP), lambda i: (0, 0)),
            pl.BlockSpec((1, HP), lambda i: (0, 0)),
            pl.BlockSpec((HP, H3P), lambda i: (0, 0)),
            pl.BlockSpec((HP, H3P), lambda i: (0, 0)),
            pl.BlockSpec((1, H3P), lambda i: (0, 0)),
            pl.BlockSpec((1, H3P), lambda i: (0, 0)),
        ],
        out_specs=pl.BlockSpec((G, HP), lambda i: (0, 0)),
        out_shape=jax.ShapeDtypeStruct((G, HP), jnp.float32),
        scratch_shapes=[
            pltpu.VMEM((1, G), jnp.float32),
            pltpu.VMEM((1, G), jnp.float32),
            pltpu.VMEM((G, HP), jnp.float32),
        ],
    )(out, xs, an, batch3, WmT, adst, mb, WihT, WhhT, bih, bhh)


def _tc_head_body(out_ref, W1T_ref, b1_ref, W2_ref, b2_ref, o_ref):
    h1 = jnp.maximum(_dot(out_ref[...], W1T_ref[...]) + b1_ref[...], 0.0)
    o_ref[...] = _dot(h1, W2_ref[...]) + b2_ref[...]


def _tc_head(out, W1T, b1, W2blk, b2):
    return pl.pallas_call(
        _tc_head_body,
        grid=(1,),
        in_specs=[
            pl.BlockSpec((G, HP), lambda i: (0, 0)),
            pl.BlockSpec((HP, 1280), lambda i: (0, 0)),
            pl.BlockSpec((1, 1280), lambda i: (0, 0)),
            pl.BlockSpec((1280, 128), lambda i: (0, 0)),
            pl.BlockSpec((1, 128), lambda i: (0, 0)),
        ],
        out_specs=pl.BlockSpec((G, 128), lambda i: (0, 0)),
        out_shape=jax.ShapeDtypeStruct((G, 128), jnp.float32),
    )(out, W1T, b1, W2blk, b2)


def _padT(W, rows, cols):
    r0, c0 = W.shape
    Wp = jnp.zeros((rows, cols), W.dtype).at[:r0, :c0].set(W)
    return Wp.T


def _padv(v, n):
    return jnp.zeros((1, n), v.dtype).at[0, :v.shape[0]].set(v)


def _pad_gru(Wih, Whh, bih, bhh):
    WihT = jnp.zeros((H3P, HP), Wih.dtype)
    WhhT = jnp.zeros((H3P, HP), Whh.dtype)
    bihp = jnp.zeros((1, H3P), bih.dtype)
    bhhp = jnp.zeros((1, H3P), bhh.dtype)
    for k in range(3):
        WihT = WihT.at[k * HP:k * HP + H, :H].set(Wih[k * H:(k + 1) * H])
        WhhT = WhhT.at[k * HP:k * HP + H, :H].set(Whh[k * H:(k + 1) * H])
        bihp = bihp.at[0, k * HP:k * HP + H].set(bih[k * H:(k + 1) * H])
        bhhp = bhhp.at[0, k * HP:k * HP + H].set(bhh[k * H:(k + 1) * H])
    return WihT.T, WhhT.T, bihp, bhhp


def kernel(x, edge_index, edge_attr, batch, lin1_W, lin1_b, gate_lin1_W,
           gate_lin2_W, gate_att_l, gate_att_r, gate_bias, gru0_Wih, gru0_Whh,
           gru0_bih, gru0_bhh, gat_W, gat_att_src, gat_att_dst, gat_bias,
           gru1_Wih, gru1_Whh, gru1_bih, gru1_bhh, mol_W, mol_att_src,
           mol_att_dst, mol_bias, mgru_Wih, mgru_Whh, mgru_bih, mgru_bhh,
           head_W1, head_b1, head_W2, head_b2):
    src = edge_index[0]
    dst = edge_index[1]
    xp = jnp.zeros((NP, FIN), jnp.float32).at[:N0].set(x)
    batchp = jnp.full((NP,), G, jnp.int32).at[:N0].set(batch)
    batch3 = batchp.reshape(NBLK, 1, 256)

    W1T = _padT(lin1_W, HP, FIN)
    b1 = _padv(lin1_b, HP)
    WaT = _padT(gate_lin1_W[:, :H], HP, HP)
    WbT = _padT(gate_lin1_W[:, H:], HP, ED)
    attl = _padv(gate_att_l, HP)
    attr_ = _padv(gate_att_r, HP)
    W2T = _padT(gate_lin2_W, HP, HP)
    gb = _padv(gate_bias, HP)
    g0 = _pad_gru(gru0_Wih, gru0_Whh, gru0_bih, gru0_bhh)
    g1 = _pad_gru(gru1_Wih, gru1_Whh, gru1_bih, gru1_bhh)
    gm = _pad_gru(mgru_Wih, mgru_Whh, mgru_bih, mgru_bhh)
    WgT = _padT(gat_W, HP, HP)
    gasrc = _padv(gat_att_src, HP)
    gadst = _padv(gat_att_dst, HP)
    gatb = _padv(gat_bias, HP)
    WmT = _padT(mol_W, HP, HP)
    masrc = _padv(mol_att_src, HP)
    madst = _padv(mol_att_dst, HP)
    mb = _padv(mol_bias, HP)
    W1r = head_W1.reshape(NT * (H // 2), H)
    hW1T = _padT(W1r, 1280, HP)
    hb1 = _padv(head_b1.reshape(-1), 1280)
    W2blk = jnp.zeros((1280, 128), jnp.float32)
    for k in range(NT):
        W2blk = W2blk.at[k * (H // 2):(k + 1) * (H // 2), k].set(head_W2[k, 0])
    hb2 = _padv(head_b2[:, 0], 128)

    zero_tile = jnp.zeros((NPT, HH), jnp.float32)

    h0, A2, d3 = _tc_node_pre(xp, W1T, b1, WaT, attr_)

    Ag2, dg = _sc_gather1(A2.reshape(2 * NP, HH), src, dst, d3.reshape(NP))
    P2 = _tc_edge1(Ag2.reshape(2, E, HH), edge_attr,
                   dg.reshape(EBLK, 1, 512), WbT, attl)
    S2 = _sc_scatter_rows(P2.reshape(2 * E, HH), dst, zero_tile)

    xc, xt2, sa3, sb3 = _tc_node_mid(S2.reshape(2, NP, HH), h0, W2T, gb,
                                     *g0, WgT, gasrc, gadst)

    X2 = _sc_gat_fused(xt2.reshape(2 * NP, HH), src, dst,
                       sa3.reshape(NP), sb3.reshape(NP), zero_tile)

    xc2, xs, an3 = _tc_node_post(X2.reshape(2, NP, HH), xc, gatb,
                                 *g1, WmT, masrc)

    out = _tc_seg_sum(xc2, batch3)
    for _ in range(2):
        out = _tc_mol_iter(out, xs, an3, batch3, WmT, madst, mb, *gm)

    logits = _tc_head(out, hW1T, hb1, W2blk, hb2)
    return logits[:, :NT]
